# Initial kernel scaffold; baseline (speedup 1.0000x reference)
import jax
import jax.numpy as jnp
from jax.experimental import pallas as pl


def kernel(x, block0_edge_index, block1_edge_index, block2_edge_index, pos_edge_index, neg_edge_index, Wself0, Wneigh0, b0, Wself1, Wneigh1, b1, Wself2, Wneigh2, b2, Wd1, bd1, Wd2, bd2, Wd3, bd3):
    raise NotImplementedError("write your pallas kernel here")


# speedup vs baseline: 2.4373x; 2.4373x over previous
import functools

import jax
import jax.numpy as jnp
from jax import lax
from jax.experimental import pallas as pl
from jax.experimental.pallas import tpu as pltpu
from jax.experimental.pallas import tpu_sc as plsc

N = 10000
E = 160000
FC = 128
TILES = 16
EDGES_PER_TILE = E // (2 * TILES) * 2
BATCH = 80
NBATCH = EDGES_PER_TILE // BATCH
SLAB = 1000
WTILES = N // SLAB
ZROWS = 50
ZCROWS = 200


def _mesh():
    return plsc.VectorSubcoreMesh(core_axis_name="c", subcore_axis_name="s")


def _zero_rows(ref, nrows, ncols):
    z = jnp.zeros((16,), jnp.float32)

    def body(i, _):
        for j in range(ncols // 16):
            ref[i, pl.ds(j * 16, 16)] = z
        return 0

    lax.fori_loop(0, nrows, body, 0, unroll=False)


def _ones_rows(ref, nrows):
    o = jnp.ones((16,), jnp.float32)

    def body(i, _):
        ref[i, pl.ds(0, 16)] = o
        return 0

    lax.fori_loop(0, nrows, body, 0, unroll=False)


def _make_sc_aggregate(nc):
    def body(*refs):
        h_chunks = refs[:nc]
        src_hbm, dst_hbm = refs[nc], refs[nc + 1]
        agg_hbm, cnt_hbm = refs[nc + 2], refs[nc + 3]
        (accum_sh, cnt_sh, idx_s, idx_d, rows, ones_b, zbuf, zcnt,
         sem) = refs[nc + 4:]

        c = lax.axis_index("c")
        s = lax.axis_index("s")
        slab = s * SLAB
        active = s < WTILES

        _zero_rows(zbuf, ZROWS, FC)
        _zero_rows(zcnt, ZCROWS, 16)
        _ones_rows(ones_b, BATCH)

        def zero_accum():
            @pl.when(active)
            def _():
                for r in range(SLAB // ZROWS):
                    pltpu.sync_copy(
                        zbuf, accum_sh.at[pl.ds(slab + r * ZROWS, ZROWS), :])

        zero_accum()

        @pl.when(active)
        def _():
            for r in range(SLAB // ZCROWS):
                pltpu.sync_copy(
                    zcnt, cnt_sh.at[pl.ds(slab + r * ZCROWS, ZCROWS), :])

        plsc.subcore_barrier()

        def chunk_loop(hk, with_counts):
            def b_body(b, _):
                off = s * EDGES_PER_TILE + b * BATCH
                pltpu.sync_copy(src_hbm.at[pl.ds(off, BATCH)], idx_s)
                pltpu.sync_copy(dst_hbm.at[pl.ds(off, BATCH)], idx_d)
                pltpu.async_copy(hk.at[idx_s], rows, sem).wait()
                pltpu.sync_copy(rows, accum_sh.at[idx_d], add=True)
                if with_counts:
                    pltpu.sync_copy(ones_b, cnt_sh.at[idx_d], add=True)
                return 0

            lax.fori_loop(0, NBATCH, b_body, 0, unroll=False)

        for rep in range(nc // 2):
            for cc in range(2):
                k = rep * 2 + cc

                @pl.when(c == cc)
                def _(k=k):
                    chunk_loop(h_chunks[k], with_counts=(k == 0))

            plsc.subcore_barrier()
            for cc in range(2):
                k = rep * 2 + cc

                @pl.when((c == cc) & active)
                def _(k=k):
                    pltpu.sync_copy(accum_sh.at[pl.ds(slab, SLAB), :],
                                    agg_hbm.at[k, pl.ds(slab, SLAB), :])

            if rep < nc // 2 - 1:
                zero_accum()
                plsc.subcore_barrier()

        @pl.when((c == 0) & active)
        def _():
            pltpu.sync_copy(cnt_sh.at[pl.ds(slab, SLAB), :],
                            cnt_hbm.at[pl.ds(slab, SLAB), :])

    return pl.kernel(
        body,
        out_type=(
            jax.ShapeDtypeStruct((nc, N, FC), jnp.float32),
            jax.ShapeDtypeStruct((N, 16), jnp.float32),
        ),
        mesh=_mesh(),
        compiler_params=pltpu.CompilerParams(use_tc_tiling_on_sc=False),
        scratch_types=[
            pltpu.VMEM_SHARED((N, FC), jnp.float32),
            pltpu.VMEM_SHARED((N, 16), jnp.float32),
            pltpu.VMEM((BATCH,), jnp.int32),
            pltpu.VMEM((BATCH,), jnp.int32),
            pltpu.VMEM((BATCH, FC), jnp.float32),
            pltpu.VMEM((BATCH, 16), jnp.float32),
            pltpu.VMEM((ZROWS, FC), jnp.float32),
            pltpu.VMEM((ZCROWS, 16), jnp.float32),
            pltpu.SemaphoreType.DMA,
        ],
    )


PAIR_ROWS = 40960
PAIR_PER_TILE = PAIR_ROWS // 32
PAIR_BATCH = 80
PAIR_NBATCH = PAIR_PER_TILE // PAIR_BATCH
H = 512


def _sc_pair_gather_body(h_hbm, se_hbm, de_hbm, es_hbm, ed_hbm,
                         idx_s, idx_d, rows_s, rows_d, sem_s, sem_d):
    c = lax.axis_index("c")
    s = lax.axis_index("s")
    w = s * 2 + c
    base = w * PAIR_PER_TILE

    def body(b, _):
        off = base + b * PAIR_BATCH
        pltpu.sync_copy(se_hbm.at[pl.ds(off, PAIR_BATCH)], idx_s)
        pltpu.sync_copy(de_hbm.at[pl.ds(off, PAIR_BATCH)], idx_d)
        cp_s = pltpu.async_copy(h_hbm.at[idx_s], rows_s, sem_s)
        cp_d = pltpu.async_copy(h_hbm.at[idx_d], rows_d, sem_d)
        cp_s.wait()
        cp_d.wait()
        pltpu.sync_copy(rows_s, es_hbm.at[pl.ds(off, PAIR_BATCH), :])
        pltpu.sync_copy(rows_d, ed_hbm.at[pl.ds(off, PAIR_BATCH), :])
        return 0

    lax.fori_loop(0, PAIR_NBATCH, body, 0, unroll=False)


def _make_sc_pair_gather():
    return pl.kernel(
        _sc_pair_gather_body,
        out_type=(
            jax.ShapeDtypeStruct((PAIR_ROWS, H), jnp.float32),
            jax.ShapeDtypeStruct((PAIR_ROWS, H), jnp.float32),
        ),
        mesh=_mesh(),
        compiler_params=pltpu.CompilerParams(use_tc_tiling_on_sc=False),
        scratch_types=[
            pltpu.VMEM((PAIR_BATCH,), jnp.int32),
            pltpu.VMEM((PAIR_BATCH,), jnp.int32),
            pltpu.VMEM((PAIR_BATCH, H), jnp.float32),
            pltpu.VMEM((PAIR_BATCH, H), jnp.float32),
            pltpu.SemaphoreType.DMA,
            pltpu.SemaphoreType.DMA,
        ],
    )


ROWS_T = 400


def _sage_tc_body(nc, relu, h_ref, agg_ref, cnt_ref, ws_ref, wn_ref, b_ref,
                  out_ref):
    recip = 1.0 / jnp.maximum(cnt_ref[:, 0:1], 1.0)
    acc = jnp.dot(h_ref[...], ws_ref[...], preferred_element_type=jnp.float32)
    for k in range(nc):
        mean_k = agg_ref[k] * recip
        acc += jnp.dot(mean_k, wn_ref[pl.ds(k * FC, FC), :],
                       preferred_element_type=jnp.float32)
    acc += b_ref[...]
    if relu:
        acc = jnp.maximum(acc, 0.0)
    out_ref[...] = acc


def _tc_sage(h, agg, cnt, wself, wneigh, b, relu):
    nin = h.shape[1]
    nc = agg.shape[0]
    grid = (N // ROWS_T,)
    return pl.pallas_call(
        functools.partial(_sage_tc_body, nc, relu),
        grid=grid,
        in_specs=[
            pl.BlockSpec((ROWS_T, nin), lambda i: (i, 0)),
            pl.BlockSpec((nc, ROWS_T, FC), lambda i: (0, i, 0)),
            pl.BlockSpec((ROWS_T, 16), lambda i: (i, 0)),
            pl.BlockSpec((nin, H), lambda i: (0, 0)),
            pl.BlockSpec((nin, H), lambda i: (0, 0)),
            pl.BlockSpec((1, H), lambda i: (0, 0)),
        ],
        out_specs=pl.BlockSpec((ROWS_T, H), lambda i: (i, 0)),
        out_shape=jax.ShapeDtypeStruct((N, H), jnp.float32),
    )(h, agg, cnt, wself, wneigh, b.reshape(1, H))


MLP_ROWS = 512


def _mlp_tc_body(es_ref, ed_ref, w1_ref, b1_ref, w2_ref, b2_ref, w3_ref,
                 b3_ref, out_ref):
    t = es_ref[...] * ed_ref[...]
    a = jnp.dot(t, w1_ref[...], preferred_element_type=jnp.float32)
    a = jnp.maximum(a + b1_ref[...], 0.0)
    a = jnp.dot(a, w2_ref[...], preferred_element_type=jnp.float32)
    a = jnp.maximum(a + b2_ref[...], 0.0)
    out_ref[...] = jnp.dot(a, w3_ref[...],
                           preferred_element_type=jnp.float32) + b3_ref[...]


def _tc_mlp(es, ed, w1, b1, w2, b2, w3p, b3p):
    grid = (PAIR_ROWS // MLP_ROWS,)
    return pl.pallas_call(
        _mlp_tc_body,
        grid=grid,
        in_specs=[
            pl.BlockSpec((MLP_ROWS, H), lambda i: (i, 0)),
            pl.BlockSpec((MLP_ROWS, H), lambda i: (i, 0)),
            pl.BlockSpec((H, H), lambda i: (0, 0)),
            pl.BlockSpec((1, H), lambda i: (0, 0)),
            pl.BlockSpec((H, H), lambda i: (0, 0)),
            pl.BlockSpec((1, H), lambda i: (0, 0)),
            pl.BlockSpec((H, 128), lambda i: (0, 0)),
            pl.BlockSpec((1, 128), lambda i: (0, 0)),
        ],
        out_specs=pl.BlockSpec((MLP_ROWS, 128), lambda i: (i, 0)),
        out_shape=jax.ShapeDtypeStruct((PAIR_ROWS, 128), jnp.float32),
    )(es, ed, w1, b1, w2, b2, w3p, b3p)


def _sage_layer(h, edge_index, wself, wneigh, b, relu):
    nin = h.shape[1]
    nc = nin // FC
    chunks = [h[:, k * FC:(k + 1) * FC] for k in range(nc)]
    agg, cnt = _make_sc_aggregate(nc)(*chunks, edge_index[0], edge_index[1])
    return _tc_sage(h, agg, cnt, wself, wneigh, b, relu)


def kernel(x, block0_edge_index, block1_edge_index, block2_edge_index,
           pos_edge_index, neg_edge_index,
           Wself0, Wneigh0, b0, Wself1, Wneigh1, b1, Wself2, Wneigh2, b2,
           Wd1, bd1, Wd2, bd2, Wd3, bd3):
    h = _sage_layer(x, block0_edge_index, Wself0, Wneigh0, b0, relu=True)
    h = _sage_layer(h, block1_edge_index, Wself1, Wneigh1, b1, relu=True)
    h = _sage_layer(h, block2_edge_index, Wself2, Wneigh2, b2, relu=False)

    pad = jnp.zeros((480,), jnp.int32)
    se = jnp.concatenate([pos_edge_index[0], pad, neg_edge_index[0], pad])
    de = jnp.concatenate([pos_edge_index[1], pad, neg_edge_index[1], pad])
    es, ed = _make_sc_pair_gather()(h, se, de)

    w3p = jnp.zeros((H, 128), jnp.float32).at[:, 0].set(Wd3[:, 0])
    b3p = jnp.zeros((1, 128), jnp.float32).at[0, 0].set(bd3[0])
    scores = _tc_mlp(es, ed, Wd1, bd1.reshape(1, H), Wd2, bd2.reshape(1, H),
                     w3p, b3p)
    h_pos = scores[:20000, 0:1]
    h_neg = scores[20480:40480, 0:1]
    return (h_pos, h_neg)


# speedup vs baseline: 4.0979x; 1.6814x over previous
import functools

import jax
import jax.numpy as jnp
from jax import lax
from jax.experimental import pallas as pl
from jax.experimental.pallas import tpu as pltpu
from jax.experimental.pallas import tpu_sc as plsc

N = 10000
E = 160000
FC = 128
TILES = 16
BATCH = 100
SUPER = 20
NSUPER = 5
SLAB = 1000
WTILES = N // SLAB


def _mesh():
    return plsc.VectorSubcoreMesh(core_axis_name="c", subcore_axis_name="s")


def _make_sc_aggregate(nc):
    def body(*refs):
        h_chunks = refs[:nc]
        src_hbm, dst_hbm = refs[nc], refs[nc + 1]
        zacc_hbm, zcnt_hbm, ones_hbm = refs[nc + 2], refs[nc + 3], refs[nc + 4]
        agg_hbm, cnt_hbm = refs[nc + 5], refs[nc + 6]
        (accum_sh, cnt_sh, src2d, dst2d, rows0, rows1, ones_b,
         sem0, sem1) = refs[nc + 7:]

        c = lax.axis_index("c")
        s = lax.axis_index("s")
        slab = s * SLAB
        active = s < WTILES
        rows_b = (rows0, rows1)
        sem_b = (sem0, sem1)

        pltpu.sync_copy(ones_hbm, ones_b)

        def zero_accum():
            @pl.when(active)
            def _():
                pltpu.sync_copy(zacc_hbm, accum_sh.at[pl.ds(slab, SLAB), :])

        zero_accum()

        @pl.when(active)
        def _():
            pltpu.sync_copy(zcnt_hbm, cnt_sh.at[pl.ds(slab, SLAB), :])

        plsc.subcore_barrier()

        def chunk_loop(hk, with_counts):
            for sb in range(NSUPER):
                pltpu.sync_copy(src_hbm.at[s, sb], src2d)
                pltpu.sync_copy(dst_hbm.at[s, sb], dst2d)
                g0 = pltpu.async_copy(hk.at[src2d.at[0]], rows0, sem0)

                def scatter(b, buf):
                    pltpu.sync_copy(buf, accum_sh.at[dst2d.at[b]], add=True)
                    if with_counts:
                        pltpu.sync_copy(ones_b, cnt_sh.at[dst2d.at[b]],
                                        add=True)

                def pair(i, _):
                    b0 = i * 2
                    b1 = b0 + 1
                    pltpu.async_copy(hk.at[src2d.at[b1]], rows1, sem1)
                    pltpu.make_async_copy(hk.at[src2d.at[b0]], rows0,
                                          sem0).wait()
                    scatter(b0, rows0)

                    @pl.when(i < SUPER // 2 - 1)
                    def _():
                        pltpu.async_copy(hk.at[src2d.at[b0 + 2]], rows0, sem0)

                    pltpu.make_async_copy(hk.at[src2d.at[b1]], rows1,
                                          sem1).wait()
                    scatter(b1, rows1)
                    return 0

                lax.fori_loop(0, SUPER // 2, pair, 0, unroll=False)
                del g0

        for rep in range(nc // 2):
            for cc in range(2):
                k = rep * 2 + cc

                @pl.when(c == cc)
                def _(k=k):
                    chunk_loop(h_chunks[k], with_counts=(k == 0))

            plsc.subcore_barrier()
            for cc in range(2):
                k = rep * 2 + cc

                @pl.when((c == cc) & active)
                def _(k=k):
                    pltpu.sync_copy(accum_sh.at[pl.ds(slab, SLAB), :],
                                    agg_hbm.at[k, pl.ds(slab, SLAB), :])

            if rep < nc // 2 - 1:
                zero_accum()
                plsc.subcore_barrier()

        @pl.when((c == 0) & active)
        def _():
            pltpu.sync_copy(cnt_sh.at[pl.ds(slab, SLAB), :],
                            cnt_hbm.at[pl.ds(slab, SLAB), :])

    return pl.kernel(
        body,
        out_type=(
            jax.ShapeDtypeStruct((nc, N, FC), jnp.float32),
            jax.ShapeDtypeStruct((N, 16), jnp.float32),
        ),
        mesh=_mesh(),
        compiler_params=pltpu.CompilerParams(use_tc_tiling_on_sc=False),
        scratch_types=[
            pltpu.VMEM_SHARED((N, FC), jnp.float32),
            pltpu.VMEM_SHARED((N, 16), jnp.float32),
            pltpu.VMEM((SUPER, BATCH), jnp.int32),
            pltpu.VMEM((SUPER, BATCH), jnp.int32),
            pltpu.VMEM((BATCH, FC), jnp.float32),
            pltpu.VMEM((BATCH, FC), jnp.float32),
            pltpu.VMEM((BATCH, 16), jnp.float32),
            pltpu.SemaphoreType.DMA,
            pltpu.SemaphoreType.DMA,
        ],
    )


PAIR_ROWS = 40960
PAIR_PER_TILE = PAIR_ROWS // 32
PAIR_BATCH = 80
PAIR_NBATCH = PAIR_PER_TILE // PAIR_BATCH
H = 512


def _sc_pair_gather_body(h_hbm, se_hbm, de_hbm, es_hbm, ed_hbm,
                         idx_s, idx_d, rows_s, rows_d, sem_s, sem_d):
    c = lax.axis_index("c")
    s = lax.axis_index("s")
    w = s * 2 + c
    base = w * PAIR_PER_TILE

    def body(b, _):
        off = base + b * PAIR_BATCH
        pltpu.sync_copy(se_hbm.at[pl.ds(off, PAIR_BATCH)], idx_s)
        pltpu.sync_copy(de_hbm.at[pl.ds(off, PAIR_BATCH)], idx_d)
        cp_s = pltpu.async_copy(h_hbm.at[idx_s], rows_s, sem_s)
        cp_d = pltpu.async_copy(h_hbm.at[idx_d], rows_d, sem_d)
        cp_s.wait()
        cp_d.wait()
        pltpu.sync_copy(rows_s, es_hbm.at[pl.ds(off, PAIR_BATCH), :])
        pltpu.sync_copy(rows_d, ed_hbm.at[pl.ds(off, PAIR_BATCH), :])
        return 0

    lax.fori_loop(0, PAIR_NBATCH, body, 0, unroll=False)


def _make_sc_pair_gather():
    return pl.kernel(
        _sc_pair_gather_body,
        out_type=(
            jax.ShapeDtypeStruct((PAIR_ROWS, H), jnp.float32),
            jax.ShapeDtypeStruct((PAIR_ROWS, H), jnp.float32),
        ),
        mesh=_mesh(),
        compiler_params=pltpu.CompilerParams(use_tc_tiling_on_sc=False),
        scratch_types=[
            pltpu.VMEM((PAIR_BATCH,), jnp.int32),
            pltpu.VMEM((PAIR_BATCH,), jnp.int32),
            pltpu.VMEM((PAIR_BATCH, H), jnp.float32),
            pltpu.VMEM((PAIR_BATCH, H), jnp.float32),
            pltpu.SemaphoreType.DMA,
            pltpu.SemaphoreType.DMA,
        ],
    )


ROWS_T = 400


def _sage_tc_body(nc, relu, h_ref, agg_ref, cnt_ref, ws_ref, wn_ref, b_ref,
                  out_ref):
    recip = 1.0 / jnp.maximum(cnt_ref[:, 0:1], 1.0)
    acc = jnp.dot(h_ref[...], ws_ref[...], preferred_element_type=jnp.float32)
    for k in range(nc):
        mean_k = agg_ref[k] * recip
        acc += jnp.dot(mean_k, wn_ref[pl.ds(k * FC, FC), :],
                       preferred_element_type=jnp.float32)
    acc += b_ref[...]
    if relu:
        acc = jnp.maximum(acc, 0.0)
    out_ref[...] = acc


def _tc_sage(h, agg, cnt, wself, wneigh, b, relu):
    nin = h.shape[1]
    nc = agg.shape[0]
    grid = (N // ROWS_T,)
    return pl.pallas_call(
        functools.partial(_sage_tc_body, nc, relu),
        grid=grid,
        in_specs=[
            pl.BlockSpec((ROWS_T, nin), lambda i: (i, 0)),
            pl.BlockSpec((nc, ROWS_T, FC), lambda i: (0, i, 0)),
            pl.BlockSpec((ROWS_T, 16), lambda i: (i, 0)),
            pl.BlockSpec((nin, H), lambda i: (0, 0)),
            pl.BlockSpec((nin, H), lambda i: (0, 0)),
            pl.BlockSpec((1, H), lambda i: (0, 0)),
        ],
        out_specs=pl.BlockSpec((ROWS_T, H), lambda i: (i, 0)),
        out_shape=jax.ShapeDtypeStruct((N, H), jnp.float32),
    )(h, agg, cnt, wself, wneigh, b.reshape(1, H))


MLP_ROWS = 512


def _mlp_tc_body(es_ref, ed_ref, w1_ref, b1_ref, w2_ref, b2_ref, w3_ref,
                 b3_ref, out_ref):
    t = es_ref[...] * ed_ref[...]
    a = jnp.dot(t, w1_ref[...], preferred_element_type=jnp.float32)
    a = jnp.maximum(a + b1_ref[...], 0.0)
    a = jnp.dot(a, w2_ref[...], preferred_element_type=jnp.float32)
    a = jnp.maximum(a + b2_ref[...], 0.0)
    out_ref[...] = jnp.dot(a, w3_ref[...],
                           preferred_element_type=jnp.float32) + b3_ref[...]


def _tc_mlp(es, ed, w1, b1, w2, b2, w3p, b3p):
    grid = (PAIR_ROWS // MLP_ROWS,)
    return pl.pallas_call(
        _mlp_tc_body,
        grid=grid,
        in_specs=[
            pl.BlockSpec((MLP_ROWS, H), lambda i: (i, 0)),
            pl.BlockSpec((MLP_ROWS, H), lambda i: (i, 0)),
            pl.BlockSpec((H, H), lambda i: (0, 0)),
            pl.BlockSpec((1, H), lambda i: (0, 0)),
            pl.BlockSpec((H, H), lambda i: (0, 0)),
            pl.BlockSpec((1, H), lambda i: (0, 0)),
            pl.BlockSpec((H, 128), lambda i: (0, 0)),
            pl.BlockSpec((1, 128), lambda i: (0, 0)),
        ],
        out_specs=pl.BlockSpec((MLP_ROWS, 128), lambda i: (i, 0)),
        out_shape=jax.ShapeDtypeStruct((PAIR_ROWS, 128), jnp.float32),
    )(es, ed, w1, b1, w2, b2, w3p, b3p)


def _sage_layer(h, edge_index, wself, wneigh, b, relu):
    nin = h.shape[1]
    nc = nin // FC
    chunks = [h[:, k * FC:(k + 1) * FC] for k in range(nc)]
    src = edge_index[0].reshape(TILES, NSUPER, SUPER, BATCH)
    dst = edge_index[1].reshape(TILES, NSUPER, SUPER, BATCH)
    zacc = jnp.zeros((SLAB, FC), jnp.float32)
    zcnt = jnp.zeros((SLAB, 16), jnp.float32)
    ones = jnp.ones((BATCH, 16), jnp.float32)
    agg, cnt = _make_sc_aggregate(nc)(*chunks, src, dst, zacc, zcnt, ones)
    return _tc_sage(h, agg, cnt, wself, wneigh, b, relu)


def kernel(x, block0_edge_index, block1_edge_index, block2_edge_index,
           pos_edge_index, neg_edge_index,
           Wself0, Wneigh0, b0, Wself1, Wneigh1, b1, Wself2, Wneigh2, b2,
           Wd1, bd1, Wd2, bd2, Wd3, bd3):
    h = _sage_layer(x, block0_edge_index, Wself0, Wneigh0, b0, relu=True)
    h = _sage_layer(h, block1_edge_index, Wself1, Wneigh1, b1, relu=True)
    h = _sage_layer(h, block2_edge_index, Wself2, Wneigh2, b2, relu=False)

    pad = jnp.zeros((480,), jnp.int32)
    se = jnp.concatenate([pos_edge_index[0], pad, neg_edge_index[0], pad])
    de = jnp.concatenate([pos_edge_index[1], pad, neg_edge_index[1], pad])
    es, ed = _make_sc_pair_gather()(h, se, de)

    w3p = jnp.zeros((H, 128), jnp.float32).at[:, 0].set(Wd3[:, 0])
    b3p = jnp.zeros((1, 128), jnp.float32).at[0, 0].set(bd3[0])
    scores = _tc_mlp(es, ed, Wd1, bd1.reshape(1, H), Wd2, bd2.reshape(1, H),
                     w3p, b3p)
    h_pos = scores[:20000, 0:1]
    h_neg = scores[20480:40480, 0:1]
    return (h_pos, h_neg)


# speedup vs baseline: 4.1962x; 1.0240x over previous
import functools

import jax
import jax.numpy as jnp
from jax import lax
from jax.experimental import pallas as pl
from jax.experimental.pallas import tpu as pltpu
from jax.experimental.pallas import tpu_sc as plsc

N = 10000
E = 160000
FC = 128
TILES = 16
BATCH = 100
SUPER = 20
NSUPER = 5
SLAB = 1000
WTILES = N // SLAB


def _mesh():
    return plsc.VectorSubcoreMesh(core_axis_name="c", subcore_axis_name="s")


def _make_sc_aggregate(nc):
    def body(*refs):
        h_chunks = refs[:nc]
        src_hbm, dst_hbm = refs[nc], refs[nc + 1]
        zacc_hbm, zcnt_hbm, ones_hbm = refs[nc + 2], refs[nc + 3], refs[nc + 4]
        agg_hbm, cnt_hbm = refs[nc + 5], refs[nc + 6]
        (accum_sh, cnt_sh, src2d, dst2d, rows0, rows1, ones_b,
         sem0, sem1) = refs[nc + 7:]

        c = lax.axis_index("c")
        s = lax.axis_index("s")
        slab = s * SLAB
        active = s < WTILES
        rows_b = (rows0, rows1)
        sem_b = (sem0, sem1)

        pltpu.sync_copy(ones_hbm, ones_b)

        def zero_accum():
            @pl.when(active)
            def _():
                pltpu.sync_copy(zacc_hbm, accum_sh.at[pl.ds(slab, SLAB), :])

        zero_accum()

        @pl.when(active)
        def _():
            pltpu.sync_copy(zcnt_hbm, cnt_sh.at[pl.ds(slab, SLAB), :])

        plsc.subcore_barrier()

        def chunk_loop(hk, with_counts):
            for sb in range(NSUPER):
                pltpu.sync_copy(src_hbm.at[s, sb], src2d)
                pltpu.sync_copy(dst_hbm.at[s, sb], dst2d)
                g0 = pltpu.async_copy(hk.at[src2d.at[0]], rows0, sem0)

                def scatter(b, buf):
                    pltpu.sync_copy(buf, accum_sh.at[dst2d.at[b]], add=True)
                    if with_counts:
                        pltpu.sync_copy(ones_b, cnt_sh.at[dst2d.at[b]],
                                        add=True)

                def pair(i, _):
                    b0 = i * 2
                    b1 = b0 + 1
                    pltpu.async_copy(hk.at[src2d.at[b1]], rows1, sem1)
                    pltpu.make_async_copy(hk.at[src2d.at[b0]], rows0,
                                          sem0).wait()
                    scatter(b0, rows0)

                    @pl.when(i < SUPER // 2 - 1)
                    def _():
                        pltpu.async_copy(hk.at[src2d.at[b0 + 2]], rows0, sem0)

                    pltpu.make_async_copy(hk.at[src2d.at[b1]], rows1,
                                          sem1).wait()
                    scatter(b1, rows1)
                    return 0

                lax.fori_loop(0, SUPER // 2, pair, 0, unroll=False)
                del g0

        for rep in range(nc // 2):
            for cc in range(2):
                k = rep * 2 + cc

                @pl.when(c == cc)
                def _(k=k):
                    chunk_loop(h_chunks[k], with_counts=(k == 0))

            plsc.subcore_barrier()
            for cc in range(2):
                k = rep * 2 + cc

                @pl.when((c == cc) & active)
                def _(k=k):
                    pltpu.sync_copy(accum_sh.at[pl.ds(slab, SLAB), :],
                                    agg_hbm.at[k, pl.ds(slab, SLAB), :])

            if rep < nc // 2 - 1:
                zero_accum()
                plsc.subcore_barrier()

        @pl.when((c == 0) & active)
        def _():
            pltpu.sync_copy(cnt_sh.at[pl.ds(slab, SLAB), :],
                            cnt_hbm.at[pl.ds(slab, SLAB), :])

    return pl.kernel(
        body,
        out_type=(
            jax.ShapeDtypeStruct((nc, N, FC), jnp.float32),
            jax.ShapeDtypeStruct((N, 16), jnp.float32),
        ),
        mesh=_mesh(),
        compiler_params=pltpu.CompilerParams(use_tc_tiling_on_sc=False),
        scratch_types=[
            pltpu.VMEM_SHARED((N, FC), jnp.float32),
            pltpu.VMEM_SHARED((N, 16), jnp.float32),
            pltpu.VMEM((SUPER, BATCH), jnp.int32),
            pltpu.VMEM((SUPER, BATCH), jnp.int32),
            pltpu.VMEM((BATCH, FC), jnp.float32),
            pltpu.VMEM((BATCH, FC), jnp.float32),
            pltpu.VMEM((BATCH, 16), jnp.float32),
            pltpu.SemaphoreType.DMA,
            pltpu.SemaphoreType.DMA,
        ],
    )


PAIR_ROWS = 40960
PAIR_PER_TILE = PAIR_ROWS // 32
PAIR_BATCH = 40
PAIR_NBATCH = PAIR_PER_TILE // PAIR_BATCH
H = 512


def _sc_pair_gather_body(h_hbm, se_hbm, de_hbm, es_hbm, ed_hbm,
                         se2d, de2d, rs0, rd0, rs1, rd1,
                         sem_s0, sem_d0, sem_s1, sem_d1):
    c = lax.axis_index("c")
    s = lax.axis_index("s")
    w = s * 2 + c
    base = w * PAIR_PER_TILE

    pltpu.sync_copy(se_hbm.at[w], se2d)
    pltpu.sync_copy(de_hbm.at[w], de2d)

    def start(b, rs, rd, ss, sd):
        pltpu.async_copy(h_hbm.at[se2d.at[b]], rs, ss)
        pltpu.async_copy(h_hbm.at[de2d.at[b]], rd, sd)

    def finish(b, rs, rd, ss, sd):
        pltpu.make_async_copy(h_hbm.at[se2d.at[b]], rs, ss).wait()
        pltpu.make_async_copy(h_hbm.at[de2d.at[b]], rd, sd).wait()
        off = base + b * PAIR_BATCH
        pltpu.sync_copy(rs, es_hbm.at[pl.ds(off, PAIR_BATCH), :])
        pltpu.sync_copy(rd, ed_hbm.at[pl.ds(off, PAIR_BATCH), :])

    start(0, rs0, rd0, sem_s0, sem_d0)

    def pair(i, _):
        b0 = i * 2
        b1 = b0 + 1
        start(b1, rs1, rd1, sem_s1, sem_d1)
        finish(b0, rs0, rd0, sem_s0, sem_d0)

        @pl.when(i < PAIR_NBATCH // 2 - 1)
        def _():
            start(b0 + 2, rs0, rd0, sem_s0, sem_d0)

        finish(b1, rs1, rd1, sem_s1, sem_d1)
        return 0

    lax.fori_loop(0, PAIR_NBATCH // 2, pair, 0, unroll=False)


def _make_sc_pair_gather():
    return pl.kernel(
        _sc_pair_gather_body,
        out_type=(
            jax.ShapeDtypeStruct((PAIR_ROWS, H), jnp.float32),
            jax.ShapeDtypeStruct((PAIR_ROWS, H), jnp.float32),
        ),
        mesh=_mesh(),
        compiler_params=pltpu.CompilerParams(use_tc_tiling_on_sc=False),
        scratch_types=[
            pltpu.VMEM((PAIR_NBATCH, PAIR_BATCH), jnp.int32),
            pltpu.VMEM((PAIR_NBATCH, PAIR_BATCH), jnp.int32),
            pltpu.VMEM((PAIR_BATCH, H), jnp.float32),
            pltpu.VMEM((PAIR_BATCH, H), jnp.float32),
            pltpu.VMEM((PAIR_BATCH, H), jnp.float32),
            pltpu.VMEM((PAIR_BATCH, H), jnp.float32),
            pltpu.SemaphoreType.DMA,
            pltpu.SemaphoreType.DMA,
            pltpu.SemaphoreType.DMA,
            pltpu.SemaphoreType.DMA,
        ],
    )


ROWS_T = 400


def _sage_tc_body(nc, relu, h_ref, agg_ref, cnt_ref, ws_ref, wn_ref, b_ref,
                  out_ref):
    recip = 1.0 / jnp.maximum(cnt_ref[:, 0:1], 1.0)
    acc = jnp.dot(h_ref[...], ws_ref[...], preferred_element_type=jnp.float32)
    for k in range(nc):
        mean_k = agg_ref[k] * recip
        acc += jnp.dot(mean_k, wn_ref[pl.ds(k * FC, FC), :],
                       preferred_element_type=jnp.float32)
    acc += b_ref[...]
    if relu:
        acc = jnp.maximum(acc, 0.0)
    out_ref[...] = acc


def _tc_sage(h, agg, cnt, wself, wneigh, b, relu):
    nin = h.shape[1]
    nc = agg.shape[0]
    grid = (N // ROWS_T,)
    return pl.pallas_call(
        functools.partial(_sage_tc_body, nc, relu),
        grid=grid,
        in_specs=[
            pl.BlockSpec((ROWS_T, nin), lambda i: (i, 0)),
            pl.BlockSpec((nc, ROWS_T, FC), lambda i: (0, i, 0)),
            pl.BlockSpec((ROWS_T, 16), lambda i: (i, 0)),
            pl.BlockSpec((nin, H), lambda i: (0, 0)),
            pl.BlockSpec((nin, H), lambda i: (0, 0)),
            pl.BlockSpec((1, H), lambda i: (0, 0)),
        ],
        out_specs=pl.BlockSpec((ROWS_T, H), lambda i: (i, 0)),
        out_shape=jax.ShapeDtypeStruct((N, H), jnp.float32),
    )(h, agg, cnt, wself, wneigh, b.reshape(1, H))


MLP_ROWS = 512


def _mlp_tc_body(es_ref, ed_ref, w1_ref, b1_ref, w2_ref, b2_ref, w3_ref,
                 b3_ref, out_ref):
    t = es_ref[...] * ed_ref[...]
    a = jnp.dot(t, w1_ref[...], preferred_element_type=jnp.float32)
    a = jnp.maximum(a + b1_ref[...], 0.0)
    a = jnp.dot(a, w2_ref[...], preferred_element_type=jnp.float32)
    a = jnp.maximum(a + b2_ref[...], 0.0)
    out_ref[...] = jnp.dot(a, w3_ref[...],
                           preferred_element_type=jnp.float32) + b3_ref[...]


def _tc_mlp(es, ed, w1, b1, w2, b2, w3p, b3p):
    grid = (PAIR_ROWS // MLP_ROWS,)
    return pl.pallas_call(
        _mlp_tc_body,
        grid=grid,
        in_specs=[
            pl.BlockSpec((MLP_ROWS, H), lambda i: (i, 0)),
            pl.BlockSpec((MLP_ROWS, H), lambda i: (i, 0)),
            pl.BlockSpec((H, H), lambda i: (0, 0)),
            pl.BlockSpec((1, H), lambda i: (0, 0)),
            pl.BlockSpec((H, H), lambda i: (0, 0)),
            pl.BlockSpec((1, H), lambda i: (0, 0)),
            pl.BlockSpec((H, 128), lambda i: (0, 0)),
            pl.BlockSpec((1, 128), lambda i: (0, 0)),
        ],
        out_specs=pl.BlockSpec((MLP_ROWS, 128), lambda i: (i, 0)),
        out_shape=jax.ShapeDtypeStruct((PAIR_ROWS, 128), jnp.float32),
    )(es, ed, w1, b1, w2, b2, w3p, b3p)


def _sage_layer(h, edge_index, wself, wneigh, b, relu):
    nin = h.shape[1]
    nc = nin // FC
    chunks = [h[:, k * FC:(k + 1) * FC] for k in range(nc)]
    src = edge_index[0].reshape(TILES, NSUPER, SUPER, BATCH)
    dst = edge_index[1].reshape(TILES, NSUPER, SUPER, BATCH)
    zacc = jnp.zeros((SLAB, FC), jnp.float32)
    zcnt = jnp.zeros((SLAB, 16), jnp.float32)
    ones = jnp.ones((BATCH, 16), jnp.float32)
    agg, cnt = _make_sc_aggregate(nc)(*chunks, src, dst, zacc, zcnt, ones)
    return _tc_sage(h, agg, cnt, wself, wneigh, b, relu)


def kernel(x, block0_edge_index, block1_edge_index, block2_edge_index,
           pos_edge_index, neg_edge_index,
           Wself0, Wneigh0, b0, Wself1, Wneigh1, b1, Wself2, Wneigh2, b2,
           Wd1, bd1, Wd2, bd2, Wd3, bd3):
    h = _sage_layer(x, block0_edge_index, Wself0, Wneigh0, b0, relu=True)
    h = _sage_layer(h, block1_edge_index, Wself1, Wneigh1, b1, relu=True)
    h = _sage_layer(h, block2_edge_index, Wself2, Wneigh2, b2, relu=False)

    pad = jnp.zeros((480,), jnp.int32)
    se = jnp.concatenate([pos_edge_index[0], pad, neg_edge_index[0], pad])
    de = jnp.concatenate([pos_edge_index[1], pad, neg_edge_index[1], pad])
    se = se.reshape(32, PAIR_NBATCH, PAIR_BATCH)
    de = de.reshape(32, PAIR_NBATCH, PAIR_BATCH)
    es, ed = _make_sc_pair_gather()(h, se, de)

    w3p = jnp.zeros((H, 128), jnp.float32).at[:, 0].set(Wd3[:, 0])
    b3p = jnp.zeros((1, 128), jnp.float32).at[0, 0].set(bd3[0])
    scores = _tc_mlp(es, ed, Wd1, bd1.reshape(1, H), Wd2, bd2.reshape(1, H),
                     w3p, b3p)
    h_pos = scores[:20000, 0:1]
    h_neg = scores[20480:40480, 0:1]
    return (h_pos, h_neg)


# speedup vs baseline: 4.1985x; 1.0005x over previous
import functools

import jax
import jax.numpy as jnp
from jax import lax
from jax.experimental import pallas as pl
from jax.experimental.pallas import tpu as pltpu
from jax.experimental.pallas import tpu_sc as plsc

N = 10000
E = 160000
FC = 128
TILES = 16
BATCH = 100
SUPER = 20
NSUPER = 5
SLAB = 1000
WTILES = N // SLAB


def _mesh():
    return plsc.VectorSubcoreMesh(core_axis_name="c", subcore_axis_name="s")


def _make_sc_aggregate(nc):
    def body(*refs):
        h_chunks = refs[:nc]
        src_hbm, dst_hbm = refs[nc], refs[nc + 1]
        zacc_hbm, zcnt_hbm, ones_hbm = refs[nc + 2], refs[nc + 3], refs[nc + 4]
        agg_hbm, cnt_hbm = refs[nc + 5], refs[nc + 6]
        (accum_sh, cnt_sh, src2d, dst2d, rows0, rows1, ones_b,
         sem0, sem1) = refs[nc + 7:]

        c = lax.axis_index("c")
        s = lax.axis_index("s")
        slab = s * SLAB
        active = s < WTILES
        rows_b = (rows0, rows1)
        sem_b = (sem0, sem1)

        pltpu.sync_copy(ones_hbm, ones_b)

        def zero_accum():
            @pl.when(active)
            def _():
                pltpu.sync_copy(zacc_hbm, accum_sh.at[pl.ds(slab, SLAB), :])

        zero_accum()

        @pl.when(active)
        def _():
            pltpu.sync_copy(zcnt_hbm, cnt_sh.at[pl.ds(slab, SLAB), :])

        plsc.subcore_barrier()

        def chunk_loop(hk, with_counts):
            for sb in range(NSUPER):
                pltpu.sync_copy(src_hbm.at[s, sb], src2d)
                pltpu.sync_copy(dst_hbm.at[s, sb], dst2d)
                g0 = pltpu.async_copy(hk.at[src2d.at[0]], rows0, sem0)

                def scatter(b, buf):
                    pltpu.sync_copy(buf, accum_sh.at[dst2d.at[b]], add=True)
                    if with_counts:
                        pltpu.sync_copy(ones_b, cnt_sh.at[dst2d.at[b]],
                                        add=True)

                def pair(i, _):
                    b0 = i * 2
                    b1 = b0 + 1
                    pltpu.async_copy(hk.at[src2d.at[b1]], rows1, sem1)
                    pltpu.make_async_copy(hk.at[src2d.at[b0]], rows0,
                                          sem0).wait()
                    scatter(b0, rows0)

                    @pl.when(i < SUPER // 2 - 1)
                    def _():
                        pltpu.async_copy(hk.at[src2d.at[b0 + 2]], rows0, sem0)

                    pltpu.make_async_copy(hk.at[src2d.at[b1]], rows1,
                                          sem1).wait()
                    scatter(b1, rows1)
                    return 0

                lax.fori_loop(0, SUPER // 2, pair, 0, unroll=False)
                del g0

        for rep in range(nc // 2):
            for cc in range(2):
                k = rep * 2 + cc

                @pl.when(c == cc)
                def _(k=k):
                    chunk_loop(h_chunks[k], with_counts=(k == 0))

            plsc.subcore_barrier()
            for cc in range(2):
                k = rep * 2 + cc

                @pl.when((c == cc) & active)
                def _(k=k):
                    pltpu.sync_copy(accum_sh.at[pl.ds(slab, SLAB), :],
                                    agg_hbm.at[k, pl.ds(slab, SLAB), :])

            if rep < nc // 2 - 1:
                zero_accum()
                plsc.subcore_barrier()

        @pl.when((c == 0) & active)
        def _():
            pltpu.sync_copy(cnt_sh.at[pl.ds(slab, SLAB), :],
                            cnt_hbm.at[pl.ds(slab, SLAB), :])

    return pl.kernel(
        body,
        out_type=(
            jax.ShapeDtypeStruct((nc, N, FC), jnp.float32),
            jax.ShapeDtypeStruct((N, 16), jnp.float32),
        ),
        mesh=_mesh(),
        compiler_params=pltpu.CompilerParams(use_tc_tiling_on_sc=False),
        scratch_types=[
            pltpu.VMEM_SHARED((N, FC), jnp.float32),
            pltpu.VMEM_SHARED((N, 16), jnp.float32),
            pltpu.VMEM((SUPER, BATCH), jnp.int32),
            pltpu.VMEM((SUPER, BATCH), jnp.int32),
            pltpu.VMEM((BATCH, FC), jnp.float32),
            pltpu.VMEM((BATCH, FC), jnp.float32),
            pltpu.VMEM((BATCH, 16), jnp.float32),
            pltpu.SemaphoreType.DMA,
            pltpu.SemaphoreType.DMA,
        ],
    )


PAIR_ROWS = 40960
PAIR_PER_TILE = PAIR_ROWS // 32
PAIR_BATCH = 40
PAIR_NBATCH = PAIR_PER_TILE // PAIR_BATCH
H = 512


def _sc_pair_gather_body(h_hbm, se_hbm, de_hbm, es_hbm, ed_hbm,
                         se2d, de2d, rs0, rd0, rs1, rd1,
                         sem_s0, sem_d0, sem_s1, sem_d1):
    c = lax.axis_index("c")
    s = lax.axis_index("s")
    w = s * 2 + c
    base = w * PAIR_PER_TILE

    pltpu.sync_copy(se_hbm.at[w], se2d)
    pltpu.sync_copy(de_hbm.at[w], de2d)

    def start(b, rs, rd, ss, sd):
        pltpu.async_copy(h_hbm.at[se2d.at[b]], rs, ss)
        pltpu.async_copy(h_hbm.at[de2d.at[b]], rd, sd)

    def finish(b, rs, rd, ss, sd):
        pltpu.make_async_copy(h_hbm.at[se2d.at[b]], rs, ss).wait()
        pltpu.make_async_copy(h_hbm.at[de2d.at[b]], rd, sd).wait()
        off = base + b * PAIR_BATCH
        pltpu.sync_copy(rs, es_hbm.at[pl.ds(off, PAIR_BATCH), :])
        pltpu.sync_copy(rd, ed_hbm.at[pl.ds(off, PAIR_BATCH), :])

    start(0, rs0, rd0, sem_s0, sem_d0)

    def pair(i, _):
        b0 = i * 2
        b1 = b0 + 1
        start(b1, rs1, rd1, sem_s1, sem_d1)
        finish(b0, rs0, rd0, sem_s0, sem_d0)

        @pl.when(i < PAIR_NBATCH // 2 - 1)
        def _():
            start(b0 + 2, rs0, rd0, sem_s0, sem_d0)

        finish(b1, rs1, rd1, sem_s1, sem_d1)
        return 0

    lax.fori_loop(0, PAIR_NBATCH // 2, pair, 0, unroll=False)


def _make_sc_pair_gather():
    return pl.kernel(
        _sc_pair_gather_body,
        out_type=(
            jax.ShapeDtypeStruct((PAIR_ROWS, H), jnp.float32),
            jax.ShapeDtypeStruct((PAIR_ROWS, H), jnp.float32),
        ),
        mesh=_mesh(),
        compiler_params=pltpu.CompilerParams(use_tc_tiling_on_sc=False),
        scratch_types=[
            pltpu.VMEM((PAIR_NBATCH, PAIR_BATCH), jnp.int32),
            pltpu.VMEM((PAIR_NBATCH, PAIR_BATCH), jnp.int32),
            pltpu.VMEM((PAIR_BATCH, H), jnp.float32),
            pltpu.VMEM((PAIR_BATCH, H), jnp.float32),
            pltpu.VMEM((PAIR_BATCH, H), jnp.float32),
            pltpu.VMEM((PAIR_BATCH, H), jnp.float32),
            pltpu.SemaphoreType.DMA,
            pltpu.SemaphoreType.DMA,
            pltpu.SemaphoreType.DMA,
            pltpu.SemaphoreType.DMA,
        ],
    )


ROWS_T = 400


def _sage_tc_body(nc, relu, h_ref, agg_ref, cnt_ref, ws_ref, wn_ref, b_ref,
                  out_ref):
    recip = 1.0 / jnp.maximum(cnt_ref[:, 0:1], 1.0)
    acc = jnp.dot(h_ref[...].astype(jnp.bfloat16),
                  ws_ref[...].astype(jnp.bfloat16),
                  preferred_element_type=jnp.float32)
    for k in range(nc):
        mean_k = (agg_ref[k] * recip).astype(jnp.bfloat16)
        acc += jnp.dot(mean_k,
                       wn_ref[pl.ds(k * FC, FC), :].astype(jnp.bfloat16),
                       preferred_element_type=jnp.float32)
    acc += b_ref[...]
    if relu:
        acc = jnp.maximum(acc, 0.0)
    out_ref[...] = acc


def _tc_sage(h, agg, cnt, wself, wneigh, b, relu):
    nin = h.shape[1]
    nc = agg.shape[0]
    grid = (N // ROWS_T,)
    return pl.pallas_call(
        functools.partial(_sage_tc_body, nc, relu),
        grid=grid,
        in_specs=[
            pl.BlockSpec((ROWS_T, nin), lambda i: (i, 0)),
            pl.BlockSpec((nc, ROWS_T, FC), lambda i: (0, i, 0)),
            pl.BlockSpec((ROWS_T, 16), lambda i: (i, 0)),
            pl.BlockSpec((nin, H), lambda i: (0, 0)),
            pl.BlockSpec((nin, H), lambda i: (0, 0)),
            pl.BlockSpec((1, H), lambda i: (0, 0)),
        ],
        out_specs=pl.BlockSpec((ROWS_T, H), lambda i: (i, 0)),
        out_shape=jax.ShapeDtypeStruct((N, H), jnp.float32),
    )(h, agg, cnt, wself, wneigh, b.reshape(1, H))


MLP_ROWS = 512


def _mlp_tc_body(es_ref, ed_ref, w1_ref, b1_ref, w2_ref, b2_ref, w3_ref,
                 b3_ref, out_ref):
    t = (es_ref[...] * ed_ref[...]).astype(jnp.bfloat16)
    a = jnp.dot(t, w1_ref[...].astype(jnp.bfloat16),
                preferred_element_type=jnp.float32)
    a = jnp.maximum(a + b1_ref[...], 0.0).astype(jnp.bfloat16)
    a = jnp.dot(a, w2_ref[...].astype(jnp.bfloat16),
                preferred_element_type=jnp.float32)
    a = jnp.maximum(a + b2_ref[...], 0.0).astype(jnp.bfloat16)
    out_ref[...] = jnp.dot(a, w3_ref[...].astype(jnp.bfloat16),
                           preferred_element_type=jnp.float32) + b3_ref[...]


def _tc_mlp(es, ed, w1, b1, w2, b2, w3p, b3p):
    grid = (PAIR_ROWS // MLP_ROWS,)
    return pl.pallas_call(
        _mlp_tc_body,
        grid=grid,
        in_specs=[
            pl.BlockSpec((MLP_ROWS, H), lambda i: (i, 0)),
            pl.BlockSpec((MLP_ROWS, H), lambda i: (i, 0)),
            pl.BlockSpec((H, H), lambda i: (0, 0)),
            pl.BlockSpec((1, H), lambda i: (0, 0)),
            pl.BlockSpec((H, H), lambda i: (0, 0)),
            pl.BlockSpec((1, H), lambda i: (0, 0)),
            pl.BlockSpec((H, 128), lambda i: (0, 0)),
            pl.BlockSpec((1, 128), lambda i: (0, 0)),
        ],
        out_specs=pl.BlockSpec((MLP_ROWS, 128), lambda i: (i, 0)),
        out_shape=jax.ShapeDtypeStruct((PAIR_ROWS, 128), jnp.float32),
    )(es, ed, w1, b1, w2, b2, w3p, b3p)


def _sage_layer(h, edge_index, wself, wneigh, b, relu):
    nin = h.shape[1]
    nc = nin // FC
    chunks = [h[:, k * FC:(k + 1) * FC] for k in range(nc)]
    src = edge_index[0].reshape(TILES, NSUPER, SUPER, BATCH)
    dst = edge_index[1].reshape(TILES, NSUPER, SUPER, BATCH)
    zacc = jnp.zeros((SLAB, FC), jnp.float32)
    zcnt = jnp.zeros((SLAB, 16), jnp.float32)
    ones = jnp.ones((BATCH, 16), jnp.float32)
    agg, cnt = _make_sc_aggregate(nc)(*chunks, src, dst, zacc, zcnt, ones)
    return _tc_sage(h, agg, cnt, wself, wneigh, b, relu)


def kernel(x, block0_edge_index, block1_edge_index, block2_edge_index,
           pos_edge_index, neg_edge_index,
           Wself0, Wneigh0, b0, Wself1, Wneigh1, b1, Wself2, Wneigh2, b2,
           Wd1, bd1, Wd2, bd2, Wd3, bd3):
    h = _sage_layer(x, block0_edge_index, Wself0, Wneigh0, b0, relu=True)
    h = _sage_layer(h, block1_edge_index, Wself1, Wneigh1, b1, relu=True)
    h = _sage_layer(h, block2_edge_index, Wself2, Wneigh2, b2, relu=False)

    pad = jnp.zeros((480,), jnp.int32)
    se = jnp.concatenate([pos_edge_index[0], pad, neg_edge_index[0], pad])
    de = jnp.concatenate([pos_edge_index[1], pad, neg_edge_index[1], pad])
    se = se.reshape(32, PAIR_NBATCH, PAIR_BATCH)
    de = de.reshape(32, PAIR_NBATCH, PAIR_BATCH)
    es, ed = _make_sc_pair_gather()(h, se, de)

    w3p = jnp.zeros((H, 128), jnp.float32).at[:, 0].set(Wd3[:, 0])
    b3p = jnp.zeros((1, 128), jnp.float32).at[0, 0].set(bd3[0])
    scores = _tc_mlp(es, ed, Wd1, bd1.reshape(1, H), Wd2, bd2.reshape(1, H),
                     w3p, b3p)
    h_pos = scores[:20000, 0:1]
    h_neg = scores[20480:40480, 0:1]
    return (h_pos, h_neg)


# speedup vs baseline: 4.6316x; 1.1032x over previous
import functools

import jax
import jax.numpy as jnp
from jax import lax
from jax.experimental import pallas as pl
from jax.experimental.pallas import tpu as pltpu
from jax.experimental.pallas import tpu_sc as plsc

N = 10000
E = 160000
FC = 128
TILES = 16
BATCH = 100
SUPER = 20
NSUPER = 5
SLAB = 1000
WTILES = N // SLAB


def _mesh():
    return plsc.VectorSubcoreMesh(core_axis_name="c", subcore_axis_name="s")


def _make_sc_aggregate(nc):
    def body(*refs):
        h_chunks = refs[:nc]
        src_hbm, dst_hbm = refs[nc], refs[nc + 1]
        zacc_hbm, zcnt_hbm, ones_hbm = refs[nc + 2], refs[nc + 3], refs[nc + 4]
        agg_hbm, cnt_hbm = refs[nc + 5], refs[nc + 6]
        (accum_sh, cnt_sh, src2d, dst2d, rows0, rows1, ones_b,
         sem0, sem1) = refs[nc + 7:]

        c = lax.axis_index("c")
        s = lax.axis_index("s")
        slab = s * SLAB
        active = s < WTILES
        rows_b = (rows0, rows1)
        sem_b = (sem0, sem1)

        pltpu.sync_copy(ones_hbm, ones_b)

        def zero_accum():
            @pl.when(active)
            def _():
                pltpu.sync_copy(zacc_hbm, accum_sh.at[pl.ds(slab, SLAB), :])

        zero_accum()

        @pl.when(active)
        def _():
            pltpu.sync_copy(zcnt_hbm, cnt_sh.at[pl.ds(slab, SLAB), :])

        plsc.subcore_barrier()

        def chunk_loop(hk, with_counts):
            for sb in range(NSUPER):
                pltpu.sync_copy(src_hbm.at[s, sb], src2d)
                pltpu.sync_copy(dst_hbm.at[s, sb], dst2d)
                g0 = pltpu.async_copy(hk.at[src2d.at[0]], rows0, sem0)

                def scatter(b, buf):
                    pltpu.sync_copy(buf, accum_sh.at[dst2d.at[b]], add=True)
                    if with_counts:
                        pltpu.sync_copy(ones_b, cnt_sh.at[dst2d.at[b]],
                                        add=True)

                def pair(i, _):
                    b0 = i * 2
                    b1 = b0 + 1
                    pltpu.async_copy(hk.at[src2d.at[b1]], rows1, sem1)
                    pltpu.make_async_copy(hk.at[src2d.at[b0]], rows0,
                                          sem0).wait()
                    scatter(b0, rows0)

                    @pl.when(i < SUPER // 2 - 1)
                    def _():
                        pltpu.async_copy(hk.at[src2d.at[b0 + 2]], rows0, sem0)

                    pltpu.make_async_copy(hk.at[src2d.at[b1]], rows1,
                                          sem1).wait()
                    scatter(b1, rows1)
                    return 0

                lax.fori_loop(0, SUPER // 2, pair, 0, unroll=False)
                del g0

        for rep in range(nc // 2):
            for cc in range(2):
                k = rep * 2 + cc

                @pl.when(c == cc)
                def _(k=k):
                    chunk_loop(h_chunks[k], with_counts=(k == 0))

            plsc.subcore_barrier()
            for cc in range(2):
                k = rep * 2 + cc

                @pl.when((c == cc) & active)
                def _(k=k):
                    pltpu.sync_copy(accum_sh.at[pl.ds(slab, SLAB), :],
                                    agg_hbm.at[k, pl.ds(slab, SLAB), :])

            if rep < nc // 2 - 1:
                zero_accum()
                plsc.subcore_barrier()

        @pl.when((c == 0) & active)
        def _():
            pltpu.sync_copy(cnt_sh.at[pl.ds(slab, SLAB), :],
                            cnt_hbm.at[pl.ds(slab, SLAB), :])

    return pl.kernel(
        body,
        out_type=(
            jax.ShapeDtypeStruct((nc, N, FC), jnp.float32),
            jax.ShapeDtypeStruct((N, 16), jnp.float32),
        ),
        mesh=_mesh(),
        compiler_params=pltpu.CompilerParams(use_tc_tiling_on_sc=False),
        scratch_types=[
            pltpu.VMEM_SHARED((N, FC), jnp.float32),
            pltpu.VMEM_SHARED((N, 16), jnp.float32),
            pltpu.VMEM((SUPER, BATCH), jnp.int32),
            pltpu.VMEM((SUPER, BATCH), jnp.int32),
            pltpu.VMEM((BATCH, FC), jnp.float32),
            pltpu.VMEM((BATCH, FC), jnp.float32),
            pltpu.VMEM((BATCH, 16), jnp.float32),
            pltpu.SemaphoreType.DMA,
            pltpu.SemaphoreType.DMA,
        ],
    )


PAIR_ROWS = 40960
PAIR_PER_TILE = PAIR_ROWS // 32
PAIR_BATCH = 40
PAIR_NBATCH = PAIR_PER_TILE // PAIR_BATCH
H = 512


def _sc_pair_gather_body(h_hbm, se_hbm, de_hbm, e_hbm,
                         se2d, de2d, rs0, rd0, rs1, rd1,
                         sem_s0, sem_d0, sem_s1, sem_d1):
    c = lax.axis_index("c")
    s = lax.axis_index("s")
    w = s * 2 + c
    base = w * PAIR_PER_TILE

    pltpu.sync_copy(se_hbm.at[w], se2d)
    pltpu.sync_copy(de_hbm.at[w], de2d)

    def start(b, rs, rd, ss, sd):
        pltpu.async_copy(h_hbm.at[se2d.at[b]], rs, ss)
        pltpu.async_copy(h_hbm.at[de2d.at[b]], rd, sd)

    def finish(b, rs, rd, ss, sd):
        pltpu.make_async_copy(h_hbm.at[se2d.at[b]], rs, ss).wait()
        pltpu.make_async_copy(h_hbm.at[de2d.at[b]], rd, sd).wait()

        def mul_row(i, _):
            for j in range(H // 16):
                sl = pl.ds(j * 16, 16)
                rs[i, sl] = rs[i, sl] * rd[i, sl]
            return 0

        lax.fori_loop(0, PAIR_BATCH, mul_row, 0, unroll=False)
        off = base + b * PAIR_BATCH
        pltpu.sync_copy(rs, e_hbm.at[pl.ds(off, PAIR_BATCH), :])

    start(0, rs0, rd0, sem_s0, sem_d0)

    def pair(i, _):
        b0 = i * 2
        b1 = b0 + 1
        start(b1, rs1, rd1, sem_s1, sem_d1)
        finish(b0, rs0, rd0, sem_s0, sem_d0)

        @pl.when(i < PAIR_NBATCH // 2 - 1)
        def _():
            start(b0 + 2, rs0, rd0, sem_s0, sem_d0)

        finish(b1, rs1, rd1, sem_s1, sem_d1)
        return 0

    lax.fori_loop(0, PAIR_NBATCH // 2, pair, 0, unroll=False)


def _make_sc_pair_gather():
    return pl.kernel(
        _sc_pair_gather_body,
        out_type=jax.ShapeDtypeStruct((PAIR_ROWS, H), jnp.float32),
        mesh=_mesh(),
        compiler_params=pltpu.CompilerParams(use_tc_tiling_on_sc=False),
        scratch_types=[
            pltpu.VMEM((PAIR_NBATCH, PAIR_BATCH), jnp.int32),
            pltpu.VMEM((PAIR_NBATCH, PAIR_BATCH), jnp.int32),
            pltpu.VMEM((PAIR_BATCH, H), jnp.float32),
            pltpu.VMEM((PAIR_BATCH, H), jnp.float32),
            pltpu.VMEM((PAIR_BATCH, H), jnp.float32),
            pltpu.VMEM((PAIR_BATCH, H), jnp.float32),
            pltpu.SemaphoreType.DMA,
            pltpu.SemaphoreType.DMA,
            pltpu.SemaphoreType.DMA,
            pltpu.SemaphoreType.DMA,
        ],
    )


ROWS_T = 400


def _sage_tc_body(nc, relu, h_ref, agg_ref, cnt_ref, ws_ref, wn_ref, b_ref,
                  out_ref):
    recip = 1.0 / jnp.maximum(cnt_ref[:, 0:1], 1.0)
    acc = jnp.dot(h_ref[...].astype(jnp.bfloat16),
                  ws_ref[...].astype(jnp.bfloat16),
                  preferred_element_type=jnp.float32)
    for k in range(nc):
        mean_k = (agg_ref[k] * recip).astype(jnp.bfloat16)
        acc += jnp.dot(mean_k,
                       wn_ref[pl.ds(k * FC, FC), :].astype(jnp.bfloat16),
                       preferred_element_type=jnp.float32)
    acc += b_ref[...]
    if relu:
        acc = jnp.maximum(acc, 0.0)
    out_ref[...] = acc


def _tc_sage(h, agg, cnt, wself, wneigh, b, relu):
    nin = h.shape[1]
    nc = agg.shape[0]
    grid = (N // ROWS_T,)
    return pl.pallas_call(
        functools.partial(_sage_tc_body, nc, relu),
        grid=grid,
        in_specs=[
            pl.BlockSpec((ROWS_T, nin), lambda i: (i, 0)),
            pl.BlockSpec((nc, ROWS_T, FC), lambda i: (0, i, 0)),
            pl.BlockSpec((ROWS_T, 16), lambda i: (i, 0)),
            pl.BlockSpec((nin, H), lambda i: (0, 0)),
            pl.BlockSpec((nin, H), lambda i: (0, 0)),
            pl.BlockSpec((1, H), lambda i: (0, 0)),
        ],
        out_specs=pl.BlockSpec((ROWS_T, H), lambda i: (i, 0)),
        out_shape=jax.ShapeDtypeStruct((N, H), jnp.float32),
    )(h, agg, cnt, wself, wneigh, b.reshape(1, H))


MLP_ROWS = 512


def _mlp_tc_body(e_ref, w1_ref, b1_ref, w2_ref, b2_ref, w3_ref,
                 b3_ref, out_ref):
    t = e_ref[...].astype(jnp.bfloat16)
    a = jnp.dot(t, w1_ref[...].astype(jnp.bfloat16),
                preferred_element_type=jnp.float32)
    a = jnp.maximum(a + b1_ref[...], 0.0).astype(jnp.bfloat16)
    a = jnp.dot(a, w2_ref[...].astype(jnp.bfloat16),
                preferred_element_type=jnp.float32)
    a = jnp.maximum(a + b2_ref[...], 0.0).astype(jnp.bfloat16)
    out_ref[...] = jnp.dot(a, w3_ref[...].astype(jnp.bfloat16),
                           preferred_element_type=jnp.float32) + b3_ref[...]


def _tc_mlp(e, w1, b1, w2, b2, w3p, b3p):
    grid = (PAIR_ROWS // MLP_ROWS,)
    return pl.pallas_call(
        _mlp_tc_body,
        grid=grid,
        in_specs=[
            pl.BlockSpec((MLP_ROWS, H), lambda i: (i, 0)),
            pl.BlockSpec((H, H), lambda i: (0, 0)),
            pl.BlockSpec((1, H), lambda i: (0, 0)),
            pl.BlockSpec((H, H), lambda i: (0, 0)),
            pl.BlockSpec((1, H), lambda i: (0, 0)),
            pl.BlockSpec((H, 128), lambda i: (0, 0)),
            pl.BlockSpec((1, 128), lambda i: (0, 0)),
        ],
        out_specs=pl.BlockSpec((MLP_ROWS, 128), lambda i: (i, 0)),
        out_shape=jax.ShapeDtypeStruct((PAIR_ROWS, 128), jnp.float32),
    )(e, w1, b1, w2, b2, w3p, b3p)


def _sage_layer(h, edge_index, wself, wneigh, b, relu):
    nin = h.shape[1]
    nc = nin // FC
    chunks = [h[:, k * FC:(k + 1) * FC] for k in range(nc)]
    src = edge_index[0].reshape(TILES, NSUPER, SUPER, BATCH)
    dst = edge_index[1].reshape(TILES, NSUPER, SUPER, BATCH)
    zacc = jnp.zeros((SLAB, FC), jnp.float32)
    zcnt = jnp.zeros((SLAB, 16), jnp.float32)
    ones = jnp.ones((BATCH, 16), jnp.float32)
    agg, cnt = _make_sc_aggregate(nc)(*chunks, src, dst, zacc, zcnt, ones)
    return _tc_sage(h, agg, cnt, wself, wneigh, b, relu)


def kernel(x, block0_edge_index, block1_edge_index, block2_edge_index,
           pos_edge_index, neg_edge_index,
           Wself0, Wneigh0, b0, Wself1, Wneigh1, b1, Wself2, Wneigh2, b2,
           Wd1, bd1, Wd2, bd2, Wd3, bd3):
    h = _sage_layer(x, block0_edge_index, Wself0, Wneigh0, b0, relu=True)
    h = _sage_layer(h, block1_edge_index, Wself1, Wneigh1, b1, relu=True)
    h = _sage_layer(h, block2_edge_index, Wself2, Wneigh2, b2, relu=False)

    pad = jnp.zeros((480,), jnp.int32)
    se = jnp.concatenate([pos_edge_index[0], pad, neg_edge_index[0], pad])
    de = jnp.concatenate([pos_edge_index[1], pad, neg_edge_index[1], pad])
    se = se.reshape(32, PAIR_NBATCH, PAIR_BATCH)
    de = de.reshape(32, PAIR_NBATCH, PAIR_BATCH)
    e = _make_sc_pair_gather()(h, se, de)

    w3p = jnp.zeros((H, 128), jnp.float32).at[:, 0].set(Wd3[:, 0])
    b3p = jnp.zeros((1, 128), jnp.float32).at[0, 0].set(bd3[0])
    scores = _tc_mlp(e, Wd1, bd1.reshape(1, H), Wd2, bd2.reshape(1, H),
                     w3p, b3p)
    h_pos = scores[:20000, 0:1]
    h_neg = scores[20480:40480, 0:1]
    return (h_pos, h_neg)


# speedup vs baseline: 4.6697x; 1.0082x over previous
import functools

import jax
import jax.numpy as jnp
from jax import lax
from jax.experimental import pallas as pl
from jax.experimental.pallas import tpu as pltpu
from jax.experimental.pallas import tpu_sc as plsc

N = 10000
E = 160000
FC = 128
TILES = 16
BATCH = 100
SUPER = 20
NSUPER = 5
SLAB = 1000
WTILES = N // SLAB


def _mesh():
    return plsc.VectorSubcoreMesh(core_axis_name="c", subcore_axis_name="s")


def _make_sc_aggregate(nc):
    def body(*refs):
        h_chunks = refs[:nc]
        src_hbm, dst_hbm = refs[nc], refs[nc + 1]
        zacc_hbm, zcnt_hbm, ones_hbm = refs[nc + 2], refs[nc + 3], refs[nc + 4]
        agg_hbm, cnt_hbm = refs[nc + 5], refs[nc + 6]
        (accum_sh, cnt_sh, src2d, dst2d, rows0, rows1, ones_b,
         sem0, sem1) = refs[nc + 7:]

        c = lax.axis_index("c")
        s = lax.axis_index("s")
        slab = s * SLAB
        active = s < WTILES
        rows_b = (rows0, rows1)
        sem_b = (sem0, sem1)

        pltpu.sync_copy(ones_hbm, ones_b)

        def zero_accum():
            @pl.when(active)
            def _():
                pltpu.sync_copy(zacc_hbm, accum_sh.at[pl.ds(slab, SLAB), :])

        zero_accum()

        @pl.when(active)
        def _():
            pltpu.sync_copy(zcnt_hbm, cnt_sh.at[pl.ds(slab, SLAB), :])

        plsc.subcore_barrier()

        def chunk_loop(hk, with_counts):
            for sb in range(NSUPER):
                pltpu.sync_copy(src_hbm.at[s, sb], src2d)
                pltpu.sync_copy(dst_hbm.at[s, sb], dst2d)
                g0 = pltpu.async_copy(hk.at[src2d.at[0]], rows0, sem0)

                def scatter(b, buf):
                    pltpu.sync_copy(buf, accum_sh.at[dst2d.at[b]], add=True)
                    if with_counts:
                        pltpu.sync_copy(ones_b, cnt_sh.at[dst2d.at[b]],
                                        add=True)

                def pair(i, _):
                    b0 = i * 2
                    b1 = b0 + 1
                    pltpu.async_copy(hk.at[src2d.at[b1]], rows1, sem1)
                    pltpu.make_async_copy(hk.at[src2d.at[b0]], rows0,
                                          sem0).wait()
                    scatter(b0, rows0)

                    @pl.when(i < SUPER // 2 - 1)
                    def _():
                        pltpu.async_copy(hk.at[src2d.at[b0 + 2]], rows0, sem0)

                    pltpu.make_async_copy(hk.at[src2d.at[b1]], rows1,
                                          sem1).wait()
                    scatter(b1, rows1)
                    return 0

                lax.fori_loop(0, SUPER // 2, pair, 0, unroll=False)
                del g0

        for rep in range(nc // 2):
            for cc in range(2):
                k = rep * 2 + cc

                @pl.when(c == cc)
                def _(k=k):
                    chunk_loop(h_chunks[k], with_counts=(k == 0))

            plsc.subcore_barrier()
            for cc in range(2):
                k = rep * 2 + cc

                @pl.when((c == cc) & active)
                def _(k=k):
                    pltpu.sync_copy(accum_sh.at[pl.ds(slab, SLAB), :],
                                    agg_hbm.at[k, pl.ds(slab, SLAB), :])

            if rep < nc // 2 - 1:
                zero_accum()
                plsc.subcore_barrier()

        @pl.when((c == 0) & active)
        def _():
            pltpu.sync_copy(cnt_sh.at[pl.ds(slab, SLAB), :],
                            cnt_hbm.at[pl.ds(slab, SLAB), :])

    return pl.kernel(
        body,
        out_type=(
            jax.ShapeDtypeStruct((nc, N, FC), jnp.float32),
            jax.ShapeDtypeStruct((N, 16), jnp.float32),
        ),
        mesh=_mesh(),
        compiler_params=pltpu.CompilerParams(use_tc_tiling_on_sc=False),
        scratch_types=[
            pltpu.VMEM_SHARED((N, FC), jnp.float32),
            pltpu.VMEM_SHARED((N, 16), jnp.float32),
            pltpu.VMEM((SUPER, BATCH), jnp.int32),
            pltpu.VMEM((SUPER, BATCH), jnp.int32),
            pltpu.VMEM((BATCH, FC), jnp.float32),
            pltpu.VMEM((BATCH, FC), jnp.float32),
            pltpu.VMEM((BATCH, 16), jnp.float32),
            pltpu.SemaphoreType.DMA,
            pltpu.SemaphoreType.DMA,
        ],
    )


PAIR_ROWS = 20480
PAIR_PER_TILE = PAIR_ROWS // 32
PAIR_BATCH = 40
PAIR_NBATCH = PAIR_PER_TILE // PAIR_BATCH
H = 512


def _sc_pair_gather_body(h_hbm, se_hbm, de_hbm, e_hbm,
                         se2d, de2d, rs0, rd0, rs1, rd1,
                         sem_s0, sem_d0, sem_s1, sem_d1):
    c = lax.axis_index("c")
    s = lax.axis_index("s")
    w = s * 2 + c
    base = w * PAIR_PER_TILE

    pltpu.sync_copy(se_hbm.at[w], se2d)
    pltpu.sync_copy(de_hbm.at[w], de2d)

    def start(b, rs, rd, ss, sd):
        pltpu.async_copy(h_hbm.at[se2d.at[b]], rs, ss)
        pltpu.async_copy(h_hbm.at[de2d.at[b]], rd, sd)

    def finish(b, rs, rd, ss, sd):
        pltpu.make_async_copy(h_hbm.at[se2d.at[b]], rs, ss).wait()
        pltpu.make_async_copy(h_hbm.at[de2d.at[b]], rd, sd).wait()

        def mul_row(i, _):
            for j in range(H // 16):
                sl = pl.ds(j * 16, 16)
                rs[i, sl] = rs[i, sl] * rd[i, sl]
            return 0

        lax.fori_loop(0, PAIR_BATCH, mul_row, 0, unroll=False)
        off = base + b * PAIR_BATCH
        pltpu.sync_copy(rs, e_hbm.at[pl.ds(off, PAIR_BATCH), :])

    start(0, rs0, rd0, sem_s0, sem_d0)

    def pair(i, _):
        b0 = i * 2
        b1 = b0 + 1
        start(b1, rs1, rd1, sem_s1, sem_d1)
        finish(b0, rs0, rd0, sem_s0, sem_d0)

        @pl.when(i < PAIR_NBATCH // 2 - 1)
        def _():
            start(b0 + 2, rs0, rd0, sem_s0, sem_d0)

        finish(b1, rs1, rd1, sem_s1, sem_d1)
        return 0

    lax.fori_loop(0, PAIR_NBATCH // 2, pair, 0, unroll=False)


def _make_sc_pair_gather():
    return pl.kernel(
        _sc_pair_gather_body,
        out_type=jax.ShapeDtypeStruct((PAIR_ROWS, H), jnp.float32),
        mesh=_mesh(),
        compiler_params=pltpu.CompilerParams(use_tc_tiling_on_sc=False),
        scratch_types=[
            pltpu.VMEM((PAIR_NBATCH, PAIR_BATCH), jnp.int32),
            pltpu.VMEM((PAIR_NBATCH, PAIR_BATCH), jnp.int32),
            pltpu.VMEM((PAIR_BATCH, H), jnp.float32),
            pltpu.VMEM((PAIR_BATCH, H), jnp.float32),
            pltpu.VMEM((PAIR_BATCH, H), jnp.float32),
            pltpu.VMEM((PAIR_BATCH, H), jnp.float32),
            pltpu.SemaphoreType.DMA,
            pltpu.SemaphoreType.DMA,
            pltpu.SemaphoreType.DMA,
            pltpu.SemaphoreType.DMA,
        ],
    )


ROWS_T = 400


def _self_tc_body(h_ref, ws_ref, b_ref, out_ref):
    out_ref[...] = jnp.dot(
        h_ref[...].astype(jnp.bfloat16), ws_ref[...].astype(jnp.bfloat16),
        preferred_element_type=jnp.float32) + b_ref[...]


def _tc_self(h, wself, b):
    nin = h.shape[1]
    grid = (N // ROWS_T,)
    return pl.pallas_call(
        _self_tc_body,
        grid=grid,
        in_specs=[
            pl.BlockSpec((ROWS_T, nin), lambda i: (i, 0)),
            pl.BlockSpec((nin, H), lambda i: (0, 0)),
            pl.BlockSpec((1, H), lambda i: (0, 0)),
        ],
        out_specs=pl.BlockSpec((ROWS_T, H), lambda i: (i, 0)),
        out_shape=jax.ShapeDtypeStruct((N, H), jnp.float32),
    )(h, wself, b.reshape(1, H))


def _combine_tc_body(nc, relu, sf_ref, agg_ref, cnt_ref, wn_ref, out_ref):
    recip = 1.0 / jnp.maximum(cnt_ref[:, 0:1], 1.0)
    acc = sf_ref[...]
    for k in range(nc):
        mean_k = (agg_ref[k] * recip).astype(jnp.bfloat16)
        acc += jnp.dot(mean_k,
                       wn_ref[pl.ds(k * FC, FC), :].astype(jnp.bfloat16),
                       preferred_element_type=jnp.float32)
    if relu:
        acc = jnp.maximum(acc, 0.0)
    out_ref[...] = acc


def _tc_sage(selfout, agg, cnt, wneigh, relu):
    nin = wneigh.shape[0]
    nc = agg.shape[0]
    grid = (N // ROWS_T,)
    return pl.pallas_call(
        functools.partial(_combine_tc_body, nc, relu),
        grid=grid,
        in_specs=[
            pl.BlockSpec((ROWS_T, H), lambda i: (i, 0)),
            pl.BlockSpec((nc, ROWS_T, FC), lambda i: (0, i, 0)),
            pl.BlockSpec((ROWS_T, 16), lambda i: (i, 0)),
            pl.BlockSpec((nin, H), lambda i: (0, 0)),
        ],
        out_specs=pl.BlockSpec((ROWS_T, H), lambda i: (i, 0)),
        out_shape=jax.ShapeDtypeStruct((N, H), jnp.float32),
    )(selfout, agg, cnt, wneigh)


MLP_ROWS = 512


def _mlp_tc_body(e_ref, w1_ref, b1_ref, w2_ref, b2_ref, w3_ref,
                 b3_ref, out_ref):
    t = e_ref[...].astype(jnp.bfloat16)
    a = jnp.dot(t, w1_ref[...].astype(jnp.bfloat16),
                preferred_element_type=jnp.float32)
    a = jnp.maximum(a + b1_ref[...], 0.0).astype(jnp.bfloat16)
    a = jnp.dot(a, w2_ref[...].astype(jnp.bfloat16),
                preferred_element_type=jnp.float32)
    a = jnp.maximum(a + b2_ref[...], 0.0).astype(jnp.bfloat16)
    out_ref[...] = jnp.dot(a, w3_ref[...].astype(jnp.bfloat16),
                           preferred_element_type=jnp.float32) + b3_ref[...]


def _tc_mlp(e, w1, b1, w2, b2, w3p, b3p):
    grid = (PAIR_ROWS // MLP_ROWS,)
    return pl.pallas_call(
        _mlp_tc_body,
        grid=grid,
        in_specs=[
            pl.BlockSpec((MLP_ROWS, H), lambda i: (i, 0)),
            pl.BlockSpec((H, H), lambda i: (0, 0)),
            pl.BlockSpec((1, H), lambda i: (0, 0)),
            pl.BlockSpec((H, H), lambda i: (0, 0)),
            pl.BlockSpec((1, H), lambda i: (0, 0)),
            pl.BlockSpec((H, 128), lambda i: (0, 0)),
            pl.BlockSpec((1, 128), lambda i: (0, 0)),
        ],
        out_specs=pl.BlockSpec((MLP_ROWS, 128), lambda i: (i, 0)),
        out_shape=jax.ShapeDtypeStruct((PAIR_ROWS, 128), jnp.float32),
    )(e, w1, b1, w2, b2, w3p, b3p)


def _sage_layer(h, edge_index, wself, wneigh, b, relu):
    nin = h.shape[1]
    nc = nin // FC
    chunks = [h[:, k * FC:(k + 1) * FC] for k in range(nc)]
    src = edge_index[0].reshape(TILES, NSUPER, SUPER, BATCH)
    dst = edge_index[1].reshape(TILES, NSUPER, SUPER, BATCH)
    zacc = jnp.zeros((SLAB, FC), jnp.float32)
    zcnt = jnp.zeros((SLAB, 16), jnp.float32)
    ones = jnp.ones((BATCH, 16), jnp.float32)
    agg, cnt = _make_sc_aggregate(nc)(*chunks, src, dst, zacc, zcnt, ones)
    selfout = _tc_self(h, wself, b)
    return _tc_sage(selfout, agg, cnt, wneigh, relu)


def kernel(x, block0_edge_index, block1_edge_index, block2_edge_index,
           pos_edge_index, neg_edge_index,
           Wself0, Wneigh0, b0, Wself1, Wneigh1, b1, Wself2, Wneigh2, b2,
           Wd1, bd1, Wd2, bd2, Wd3, bd3):
    h = _sage_layer(x, block0_edge_index, Wself0, Wneigh0, b0, relu=True)
    h = _sage_layer(h, block1_edge_index, Wself1, Wneigh1, b1, relu=True)
    h = _sage_layer(h, block2_edge_index, Wself2, Wneigh2, b2, relu=False)

    pad = jnp.zeros((480,), jnp.int32)
    gather = _make_sc_pair_gather()
    w3p = jnp.zeros((H, 128), jnp.float32).at[:, 0].set(Wd3[:, 0])
    b3p = jnp.zeros((1, 128), jnp.float32).at[0, 0].set(bd3[0])

    def half(ei):
        se = jnp.concatenate([ei[0], pad]).reshape(32, PAIR_NBATCH, PAIR_BATCH)
        de = jnp.concatenate([ei[1], pad]).reshape(32, PAIR_NBATCH, PAIR_BATCH)
        e = gather(h, se, de)
        return _tc_mlp(e, Wd1, bd1.reshape(1, H), Wd2, bd2.reshape(1, H),
                       w3p, b3p)

    h_pos = half(pos_edge_index)[:20000, 0:1]
    h_neg = half(neg_edge_index)[:20000, 0:1]
    return (h_pos, h_neg)


# speedup vs baseline: 4.7419x; 1.0155x over previous
import functools

import jax
import jax.numpy as jnp
from jax import lax
from jax.experimental import pallas as pl
from jax.experimental.pallas import tpu as pltpu
from jax.experimental.pallas import tpu_sc as plsc

N = 10000
E = 160000
FC = 128
TILES = 16
BATCH = 125
SUPER = 16
NSUPER = 5
SLAB = 1000
WTILES = N // SLAB


def _mesh():
    return plsc.VectorSubcoreMesh(core_axis_name="c", subcore_axis_name="s")


def _make_sc_aggregate(nc):
    def body(*refs):
        h_chunks = refs[:nc]
        src_hbm, dst_hbm = refs[nc], refs[nc + 1]
        zacc_hbm, zcnt_hbm, ones_hbm = refs[nc + 2], refs[nc + 3], refs[nc + 4]
        agg_hbm, cnt_hbm = refs[nc + 5], refs[nc + 6]
        (accum_sh, cnt_sh, src2d, dst2d, rows0, rows1, ones_b,
         sem0, sem1) = refs[nc + 7:]

        c = lax.axis_index("c")
        s = lax.axis_index("s")
        slab = s * SLAB
        active = s < WTILES
        rows_b = (rows0, rows1)
        sem_b = (sem0, sem1)

        pltpu.sync_copy(ones_hbm, ones_b)

        def zero_accum():
            @pl.when(active)
            def _():
                pltpu.sync_copy(zacc_hbm, accum_sh.at[pl.ds(slab, SLAB), :])

        zero_accum()

        @pl.when(active)
        def _():
            pltpu.sync_copy(zcnt_hbm, cnt_sh.at[pl.ds(slab, SLAB), :])

        plsc.subcore_barrier()

        def chunk_loop(hk, with_counts):
            for sb in range(NSUPER):
                pltpu.sync_copy(src_hbm.at[s, sb], src2d)
                pltpu.sync_copy(dst_hbm.at[s, sb], dst2d)
                g0 = pltpu.async_copy(hk.at[src2d.at[0]], rows0, sem0)

                def scatter(b, buf):
                    pltpu.sync_copy(buf, accum_sh.at[dst2d.at[b]], add=True)
                    if with_counts:
                        pltpu.sync_copy(ones_b, cnt_sh.at[dst2d.at[b]],
                                        add=True)

                def pair(i, _):
                    b0 = i * 2
                    b1 = b0 + 1
                    pltpu.async_copy(hk.at[src2d.at[b1]], rows1, sem1)
                    pltpu.make_async_copy(hk.at[src2d.at[b0]], rows0,
                                          sem0).wait()
                    scatter(b0, rows0)

                    @pl.when(i < SUPER // 2 - 1)
                    def _():
                        pltpu.async_copy(hk.at[src2d.at[b0 + 2]], rows0, sem0)

                    pltpu.make_async_copy(hk.at[src2d.at[b1]], rows1,
                                          sem1).wait()
                    scatter(b1, rows1)
                    return 0

                lax.fori_loop(0, SUPER // 2, pair, 0, unroll=False)
                del g0

        for rep in range(nc // 2):
            for cc in range(2):
                k = rep * 2 + cc

                @pl.when(c == cc)
                def _(k=k):
                    chunk_loop(h_chunks[k], with_counts=(k == 0))

            plsc.subcore_barrier()
            for cc in range(2):
                k = rep * 2 + cc

                @pl.when((c == cc) & active)
                def _(k=k):
                    pltpu.sync_copy(accum_sh.at[pl.ds(slab, SLAB), :],
                                    agg_hbm.at[k, pl.ds(slab, SLAB), :])

            if rep < nc // 2 - 1:
                zero_accum()
                plsc.subcore_barrier()

        @pl.when((c == 0) & active)
        def _():
            pltpu.sync_copy(cnt_sh.at[pl.ds(slab, SLAB), :],
                            cnt_hbm.at[pl.ds(slab, SLAB), :])

    return pl.kernel(
        body,
        out_type=(
            jax.ShapeDtypeStruct((nc, N, FC), jnp.float32),
            jax.ShapeDtypeStruct((N, 16), jnp.float32),
        ),
        mesh=_mesh(),
        compiler_params=pltpu.CompilerParams(use_tc_tiling_on_sc=False),
        scratch_types=[
            pltpu.VMEM_SHARED((N, FC), jnp.float32),
            pltpu.VMEM_SHARED((N, 16), jnp.float32),
            pltpu.VMEM((SUPER, BATCH), jnp.int32),
            pltpu.VMEM((SUPER, BATCH), jnp.int32),
            pltpu.VMEM((BATCH, FC), jnp.float32),
            pltpu.VMEM((BATCH, FC), jnp.float32),
            pltpu.VMEM((BATCH, 16), jnp.float32),
            pltpu.SemaphoreType.DMA,
            pltpu.SemaphoreType.DMA,
        ],
    )


PAIR_ROWS = 20480
PAIR_PER_TILE = PAIR_ROWS // 32
PAIR_BATCH = 40
PAIR_NBATCH = PAIR_PER_TILE // PAIR_BATCH
H = 512


def _sc_pair_gather_body(h_hbm, se_hbm, de_hbm, e_hbm,
                         se2d, de2d, rs0, rd0, rs1, rd1,
                         sem_s0, sem_d0, sem_s1, sem_d1):
    c = lax.axis_index("c")
    s = lax.axis_index("s")
    w = s * 2 + c
    base = w * PAIR_PER_TILE

    pltpu.sync_copy(se_hbm.at[w], se2d)
    pltpu.sync_copy(de_hbm.at[w], de2d)

    def start(b, rs, rd, ss, sd):
        pltpu.async_copy(h_hbm.at[se2d.at[b]], rs, ss)
        pltpu.async_copy(h_hbm.at[de2d.at[b]], rd, sd)

    def finish(b, rs, rd, ss, sd):
        pltpu.make_async_copy(h_hbm.at[se2d.at[b]], rs, ss).wait()
        pltpu.make_async_copy(h_hbm.at[de2d.at[b]], rd, sd).wait()

        def mul_row(i, _):
            for j in range(H // 16):
                sl = pl.ds(j * 16, 16)
                rs[i, sl] = rs[i, sl] * rd[i, sl]
            return 0

        lax.fori_loop(0, PAIR_BATCH, mul_row, 0, unroll=False)
        off = base + b * PAIR_BATCH
        pltpu.sync_copy(rs, e_hbm.at[pl.ds(off, PAIR_BATCH), :])

    start(0, rs0, rd0, sem_s0, sem_d0)

    def pair(i, _):
        b0 = i * 2
        b1 = b0 + 1
        start(b1, rs1, rd1, sem_s1, sem_d1)
        finish(b0, rs0, rd0, sem_s0, sem_d0)

        @pl.when(i < PAIR_NBATCH // 2 - 1)
        def _():
            start(b0 + 2, rs0, rd0, sem_s0, sem_d0)

        finish(b1, rs1, rd1, sem_s1, sem_d1)
        return 0

    lax.fori_loop(0, PAIR_NBATCH // 2, pair, 0, unroll=False)


def _make_sc_pair_gather():
    return pl.kernel(
        _sc_pair_gather_body,
        out_type=jax.ShapeDtypeStruct((PAIR_ROWS, H), jnp.float32),
        mesh=_mesh(),
        compiler_params=pltpu.CompilerParams(use_tc_tiling_on_sc=False),
        scratch_types=[
            pltpu.VMEM((PAIR_NBATCH, PAIR_BATCH), jnp.int32),
            pltpu.VMEM((PAIR_NBATCH, PAIR_BATCH), jnp.int32),
            pltpu.VMEM((PAIR_BATCH, H), jnp.float32),
            pltpu.VMEM((PAIR_BATCH, H), jnp.float32),
            pltpu.VMEM((PAIR_BATCH, H), jnp.float32),
            pltpu.VMEM((PAIR_BATCH, H), jnp.float32),
            pltpu.SemaphoreType.DMA,
            pltpu.SemaphoreType.DMA,
            pltpu.SemaphoreType.DMA,
            pltpu.SemaphoreType.DMA,
        ],
    )


ROWS_T = 400


def _self_tc_body(h_ref, ws_ref, b_ref, out_ref):
    out_ref[...] = jnp.dot(
        h_ref[...].astype(jnp.bfloat16), ws_ref[...].astype(jnp.bfloat16),
        preferred_element_type=jnp.float32) + b_ref[...]


def _tc_self(h, wself, b):
    nin = h.shape[1]
    grid = (N // ROWS_T,)
    return pl.pallas_call(
        _self_tc_body,
        grid=grid,
        in_specs=[
            pl.BlockSpec((ROWS_T, nin), lambda i: (i, 0)),
            pl.BlockSpec((nin, H), lambda i: (0, 0)),
            pl.BlockSpec((1, H), lambda i: (0, 0)),
        ],
        out_specs=pl.BlockSpec((ROWS_T, H), lambda i: (i, 0)),
        out_shape=jax.ShapeDtypeStruct((N, H), jnp.float32),
    )(h, wself, b.reshape(1, H))


def _combine_tc_body(nc, relu, sf_ref, agg_ref, cnt_ref, wn_ref, out_ref):
    recip = 1.0 / jnp.maximum(cnt_ref[:, 0:1], 1.0)
    acc = sf_ref[...]
    for k in range(nc):
        mean_k = (agg_ref[k] * recip).astype(jnp.bfloat16)
        acc += jnp.dot(mean_k,
                       wn_ref[pl.ds(k * FC, FC), :].astype(jnp.bfloat16),
                       preferred_element_type=jnp.float32)
    if relu:
        acc = jnp.maximum(acc, 0.0)
    out_ref[...] = acc


def _tc_sage(selfout, agg, cnt, wneigh, relu):
    nin = wneigh.shape[0]
    nc = agg.shape[0]
    grid = (N // ROWS_T,)
    return pl.pallas_call(
        functools.partial(_combine_tc_body, nc, relu),
        grid=grid,
        in_specs=[
            pl.BlockSpec((ROWS_T, H), lambda i: (i, 0)),
            pl.BlockSpec((nc, ROWS_T, FC), lambda i: (0, i, 0)),
            pl.BlockSpec((ROWS_T, 16), lambda i: (i, 0)),
            pl.BlockSpec((nin, H), lambda i: (0, 0)),
        ],
        out_specs=pl.BlockSpec((ROWS_T, H), lambda i: (i, 0)),
        out_shape=jax.ShapeDtypeStruct((N, H), jnp.float32),
    )(selfout, agg, cnt, wneigh)


MLP_ROWS = 512


def _mlp_tc_body(e_ref, w1_ref, b1_ref, w2_ref, b2_ref, w3_ref,
                 b3_ref, out_ref):
    t = e_ref[...].astype(jnp.bfloat16)
    a = jnp.dot(t, w1_ref[...].astype(jnp.bfloat16),
                preferred_element_type=jnp.float32)
    a = jnp.maximum(a + b1_ref[...], 0.0).astype(jnp.bfloat16)
    a = jnp.dot(a, w2_ref[...].astype(jnp.bfloat16),
                preferred_element_type=jnp.float32)
    a = jnp.maximum(a + b2_ref[...], 0.0).astype(jnp.bfloat16)
    out_ref[...] = jnp.dot(a, w3_ref[...].astype(jnp.bfloat16),
                           preferred_element_type=jnp.float32) + b3_ref[...]


def _tc_mlp(e, w1, b1, w2, b2, w3p, b3p):
    grid = (PAIR_ROWS // MLP_ROWS,)
    return pl.pallas_call(
        _mlp_tc_body,
        grid=grid,
        in_specs=[
            pl.BlockSpec((MLP_ROWS, H), lambda i: (i, 0)),
            pl.BlockSpec((H, H), lambda i: (0, 0)),
            pl.BlockSpec((1, H), lambda i: (0, 0)),
            pl.BlockSpec((H, H), lambda i: (0, 0)),
            pl.BlockSpec((1, H), lambda i: (0, 0)),
            pl.BlockSpec((H, 128), lambda i: (0, 0)),
            pl.BlockSpec((1, 128), lambda i: (0, 0)),
        ],
        out_specs=pl.BlockSpec((MLP_ROWS, 128), lambda i: (i, 0)),
        out_shape=jax.ShapeDtypeStruct((PAIR_ROWS, 128), jnp.float32),
    )(e, w1, b1, w2, b2, w3p, b3p)


def _sage_layer(h, edge_index, wself, wneigh, b, relu):
    nin = h.shape[1]
    nc = nin // FC
    chunks = [h[:, k * FC:(k + 1) * FC] for k in range(nc)]
    src = edge_index[0].reshape(TILES, NSUPER, SUPER, BATCH)
    dst = edge_index[1].reshape(TILES, NSUPER, SUPER, BATCH)
    zacc = jnp.zeros((SLAB, FC), jnp.float32)
    zcnt = jnp.zeros((SLAB, 16), jnp.float32)
    ones = jnp.ones((BATCH, 16), jnp.float32)
    agg, cnt = _make_sc_aggregate(nc)(*chunks, src, dst, zacc, zcnt, ones)
    selfout = _tc_self(h, wself, b)
    return _tc_sage(selfout, agg, cnt, wneigh, relu)


def kernel(x, block0_edge_index, block1_edge_index, block2_edge_index,
           pos_edge_index, neg_edge_index,
           Wself0, Wneigh0, b0, Wself1, Wneigh1, b1, Wself2, Wneigh2, b2,
           Wd1, bd1, Wd2, bd2, Wd3, bd3):
    h = _sage_layer(x, block0_edge_index, Wself0, Wneigh0, b0, relu=True)
    h = _sage_layer(h, block1_edge_index, Wself1, Wneigh1, b1, relu=True)
    h = _sage_layer(h, block2_edge_index, Wself2, Wneigh2, b2, relu=False)

    pad = jnp.zeros((480,), jnp.int32)
    gather = _make_sc_pair_gather()
    w3p = jnp.zeros((H, 128), jnp.float32).at[:, 0].set(Wd3[:, 0])
    b3p = jnp.zeros((1, 128), jnp.float32).at[0, 0].set(bd3[0])

    def half(ei):
        se = jnp.concatenate([ei[0], pad]).reshape(32, PAIR_NBATCH, PAIR_BATCH)
        de = jnp.concatenate([ei[1], pad]).reshape(32, PAIR_NBATCH, PAIR_BATCH)
        e = gather(h, se, de)
        return _tc_mlp(e, Wd1, bd1.reshape(1, H), Wd2, bd2.reshape(1, H),
                       w3p, b3p)

    h_pos = half(pos_edge_index)[:20000, 0:1]
    h_neg = half(neg_edge_index)[:20000, 0:1]
    return (h_pos, h_neg)


# speedup vs baseline: 4.8625x; 1.0254x over previous
import functools

import jax
import jax.numpy as jnp
from jax import lax
from jax.experimental import pallas as pl
from jax.experimental.pallas import tpu as pltpu
from jax.experimental.pallas import tpu_sc as plsc

N = 10000
E = 160000
FC = 128
TILES = 16
BATCH = 125
SUPER = 16
NSUPER = 5
SLAB = 1000
WTILES = N // SLAB


def _mesh():
    return plsc.VectorSubcoreMesh(core_axis_name="c", subcore_axis_name="s")


def _make_sc_aggregate(nc):
    def body(*refs):
        h_chunks = refs[:nc]
        src_hbm, dst_hbm = refs[nc], refs[nc + 1]
        zacc_hbm, zcnt_hbm, ones_hbm = refs[nc + 2], refs[nc + 3], refs[nc + 4]
        agg_hbm, cnt_hbm = refs[nc + 5], refs[nc + 6]
        (accum_sh, cnt_sh, src2d, dst2d, rows0, rows1, ones_b,
         sem0, sem1) = refs[nc + 7:]

        c = lax.axis_index("c")
        s = lax.axis_index("s")
        slab = s * SLAB
        active = s < WTILES
        rows_b = (rows0, rows1)
        sem_b = (sem0, sem1)

        pltpu.sync_copy(ones_hbm, ones_b)

        def zero_accum():
            @pl.when(active)
            def _():
                pltpu.sync_copy(zacc_hbm, accum_sh.at[pl.ds(slab, SLAB), :])

        zero_accum()

        @pl.when(active)
        def _():
            pltpu.sync_copy(zcnt_hbm, cnt_sh.at[pl.ds(slab, SLAB), :])

        plsc.subcore_barrier()

        def chunk_loop(hk, with_counts):
            for sb in range(NSUPER):
                pltpu.sync_copy(src_hbm.at[s, sb], src2d)
                pltpu.sync_copy(dst_hbm.at[s, sb], dst2d)
                g0 = pltpu.async_copy(hk.at[src2d.at[0]], rows0, sem0)

                def scatter(b, buf):
                    pltpu.sync_copy(buf, accum_sh.at[dst2d.at[b]], add=True)
                    if with_counts:
                        pltpu.sync_copy(ones_b, cnt_sh.at[dst2d.at[b]],
                                        add=True)

                def pair(i, _):
                    b0 = i * 2
                    b1 = b0 + 1
                    pltpu.async_copy(hk.at[src2d.at[b1]], rows1, sem1)
                    pltpu.make_async_copy(hk.at[src2d.at[b0]], rows0,
                                          sem0).wait()
                    scatter(b0, rows0)

                    @pl.when(i < SUPER // 2 - 1)
                    def _():
                        pltpu.async_copy(hk.at[src2d.at[b0 + 2]], rows0, sem0)

                    pltpu.make_async_copy(hk.at[src2d.at[b1]], rows1,
                                          sem1).wait()
                    scatter(b1, rows1)
                    return 0

                lax.fori_loop(0, SUPER // 2, pair, 0, unroll=False)
                del g0

        for rep in range(nc // 2):
            for cc in range(2):
                k = rep * 2 + cc

                @pl.when(c == cc)
                def _(k=k):
                    chunk_loop(h_chunks[k], with_counts=(k == 0))

            plsc.subcore_barrier()
            for cc in range(2):
                k = rep * 2 + cc

                @pl.when((c == cc) & active)
                def _(k=k):
                    pltpu.sync_copy(accum_sh.at[pl.ds(slab, SLAB), :],
                                    agg_hbm.at[k, pl.ds(slab, SLAB), :])

            if rep < nc // 2 - 1:
                zero_accum()
                plsc.subcore_barrier()

        @pl.when((c == 0) & active)
        def _():
            pltpu.sync_copy(cnt_sh.at[pl.ds(slab, SLAB), :],
                            cnt_hbm.at[pl.ds(slab, SLAB), :])

    return pl.kernel(
        body,
        out_type=(
            jax.ShapeDtypeStruct((nc, N, FC), jnp.float32),
            jax.ShapeDtypeStruct((N, 16), jnp.float32),
        ),
        mesh=_mesh(),
        compiler_params=pltpu.CompilerParams(use_tc_tiling_on_sc=False),
        scratch_types=[
            pltpu.VMEM_SHARED((N, FC), jnp.float32),
            pltpu.VMEM_SHARED((N, 16), jnp.float32),
            pltpu.VMEM((SUPER, BATCH), jnp.int32),
            pltpu.VMEM((SUPER, BATCH), jnp.int32),
            pltpu.VMEM((BATCH, FC), jnp.float32),
            pltpu.VMEM((BATCH, FC), jnp.float32),
            pltpu.VMEM((BATCH, 16), jnp.float32),
            pltpu.SemaphoreType.DMA,
            pltpu.SemaphoreType.DMA,
        ],
    )


PAIR_ROWS = 20480
PAIR_PER_TILE = PAIR_ROWS // 32
PAIR_BATCH = 40
PAIR_NBATCH = PAIR_PER_TILE // PAIR_BATCH
H = 512


def _sc_pair_gather_body(h_hbm, se_hbm, de_hbm, e_hbm,
                         se2d, de2d, rs0, rd0, rs1, rd1,
                         sem_s0, sem_d0, sem_s1, sem_d1):
    c = lax.axis_index("c")
    s = lax.axis_index("s")
    w = s * 2 + c
    base = w * PAIR_PER_TILE

    pltpu.sync_copy(se_hbm.at[w], se2d)
    pltpu.sync_copy(de_hbm.at[w], de2d)

    def start(b, rs, rd, ss, sd):
        pltpu.async_copy(h_hbm.at[se2d.at[b]], rs, ss)
        pltpu.async_copy(h_hbm.at[de2d.at[b]], rd, sd)

    def finish(b, rs, rd, ss, sd):
        pltpu.make_async_copy(h_hbm.at[se2d.at[b]], rs, ss).wait()
        pltpu.make_async_copy(h_hbm.at[de2d.at[b]], rd, sd).wait()

        def mul_row(i, _):
            for j in range(H // 16):
                sl = pl.ds(j * 16, 16)
                rs[i, sl] = rs[i, sl] * rd[i, sl]
            return 0

        lax.fori_loop(0, PAIR_BATCH, mul_row, 0, unroll=False)
        off = base + b * PAIR_BATCH
        pltpu.sync_copy(rs, e_hbm.at[pl.ds(off, PAIR_BATCH), :])

    start(0, rs0, rd0, sem_s0, sem_d0)

    def pair(i, _):
        b0 = i * 2
        b1 = b0 + 1
        start(b1, rs1, rd1, sem_s1, sem_d1)
        finish(b0, rs0, rd0, sem_s0, sem_d0)

        @pl.when(i < PAIR_NBATCH // 2 - 1)
        def _():
            start(b0 + 2, rs0, rd0, sem_s0, sem_d0)

        finish(b1, rs1, rd1, sem_s1, sem_d1)
        return 0

    lax.fori_loop(0, PAIR_NBATCH // 2, pair, 0, unroll=False)


def _make_sc_pair_gather():
    return pl.kernel(
        _sc_pair_gather_body,
        out_type=jax.ShapeDtypeStruct((PAIR_ROWS, H), jnp.float32),
        mesh=_mesh(),
        compiler_params=pltpu.CompilerParams(use_tc_tiling_on_sc=False),
        scratch_types=[
            pltpu.VMEM((PAIR_NBATCH, PAIR_BATCH), jnp.int32),
            pltpu.VMEM((PAIR_NBATCH, PAIR_BATCH), jnp.int32),
            pltpu.VMEM((PAIR_BATCH, H), jnp.float32),
            pltpu.VMEM((PAIR_BATCH, H), jnp.float32),
            pltpu.VMEM((PAIR_BATCH, H), jnp.float32),
            pltpu.VMEM((PAIR_BATCH, H), jnp.float32),
            pltpu.SemaphoreType.DMA,
            pltpu.SemaphoreType.DMA,
            pltpu.SemaphoreType.DMA,
            pltpu.SemaphoreType.DMA,
        ],
    )


ROWS_T = 400


def _self_tc_body(nc, ws_ref, b_ref, *refs):
    chunk_refs, out_ref = refs[:nc], refs[nc]
    acc = b_ref[...].astype(jnp.float32) + jnp.zeros((ROWS_T, H), jnp.float32)
    for k in range(nc):
        acc += jnp.dot(chunk_refs[k][...].astype(jnp.bfloat16),
                       ws_ref[pl.ds(k * FC, FC), :].astype(jnp.bfloat16),
                       preferred_element_type=jnp.float32)
    out_ref[...] = acc


def _tc_self(chunks, wself, b):
    nin = wself.shape[0]
    nc = len(chunks)
    grid = (N // ROWS_T,)
    return pl.pallas_call(
        functools.partial(_self_tc_body, nc),
        grid=grid,
        in_specs=[
            pl.BlockSpec((nin, H), lambda i: (0, 0)),
            pl.BlockSpec((1, H), lambda i: (0, 0)),
        ] + [pl.BlockSpec((ROWS_T, FC), lambda i: (i, 0))] * nc,
        out_specs=pl.BlockSpec((ROWS_T, H), lambda i: (i, 0)),
        out_shape=jax.ShapeDtypeStruct((N, H), jnp.float32),
    )(wself, b.reshape(1, H), *chunks)


def _combine_tc_body(nc, relu, chunked, sf_ref, agg_ref, cnt_ref, wn_ref,
                     *out_refs):
    recip = 1.0 / jnp.maximum(cnt_ref[:, 0:1], 1.0)
    acc = sf_ref[...]
    for k in range(nc):
        mean_k = (agg_ref[k] * recip).astype(jnp.bfloat16)
        acc += jnp.dot(mean_k,
                       wn_ref[pl.ds(k * FC, FC), :].astype(jnp.bfloat16),
                       preferred_element_type=jnp.float32)
    if relu:
        acc = jnp.maximum(acc, 0.0)
    if chunked:
        for k in range(H // FC):
            out_refs[k][...] = acc[:, k * FC:(k + 1) * FC]
    else:
        out_refs[0][...] = acc


def _tc_sage(selfout, agg, cnt, wneigh, relu, chunked):
    nin = wneigh.shape[0]
    nc = agg.shape[0]
    grid = (N // ROWS_T,)
    if chunked:
        out_specs = [pl.BlockSpec((ROWS_T, FC), lambda i: (i, 0))] * (H // FC)
        out_shape = [jax.ShapeDtypeStruct((N, FC), jnp.float32)] * (H // FC)
    else:
        out_specs = [pl.BlockSpec((ROWS_T, H), lambda i: (i, 0))]
        out_shape = [jax.ShapeDtypeStruct((N, H), jnp.float32)]
    return pl.pallas_call(
        functools.partial(_combine_tc_body, nc, relu, chunked),
        grid=grid,
        in_specs=[
            pl.BlockSpec((ROWS_T, H), lambda i: (i, 0)),
            pl.BlockSpec((nc, ROWS_T, FC), lambda i: (0, i, 0)),
            pl.BlockSpec((ROWS_T, 16), lambda i: (i, 0)),
            pl.BlockSpec((nin, H), lambda i: (0, 0)),
        ],
        out_specs=out_specs,
        out_shape=out_shape,
    )(selfout, agg, cnt, wneigh)


MLP_ROWS = 512


def _mlp_tc_body(e_ref, w1_ref, b1_ref, w2_ref, b2_ref, w3_ref,
                 b3_ref, out_ref):
    t = e_ref[...].astype(jnp.bfloat16)
    a = jnp.dot(t, w1_ref[...].astype(jnp.bfloat16),
                preferred_element_type=jnp.float32)
    a = jnp.maximum(a + b1_ref[...], 0.0).astype(jnp.bfloat16)
    a = jnp.dot(a, w2_ref[...].astype(jnp.bfloat16),
                preferred_element_type=jnp.float32)
    a = jnp.maximum(a + b2_ref[...], 0.0).astype(jnp.bfloat16)
    out_ref[...] = jnp.dot(a, w3_ref[...].astype(jnp.bfloat16),
                           preferred_element_type=jnp.float32) + b3_ref[...]


def _tc_mlp(e, w1, b1, w2, b2, w3p, b3p):
    grid = (PAIR_ROWS // MLP_ROWS,)
    return pl.pallas_call(
        _mlp_tc_body,
        grid=grid,
        in_specs=[
            pl.BlockSpec((MLP_ROWS, H), lambda i: (i, 0)),
            pl.BlockSpec((H, H), lambda i: (0, 0)),
            pl.BlockSpec((1, H), lambda i: (0, 0)),
            pl.BlockSpec((H, H), lambda i: (0, 0)),
            pl.BlockSpec((1, H), lambda i: (0, 0)),
            pl.BlockSpec((H, 128), lambda i: (0, 0)),
            pl.BlockSpec((1, 128), lambda i: (0, 0)),
        ],
        out_specs=pl.BlockSpec((MLP_ROWS, 128), lambda i: (i, 0)),
        out_shape=jax.ShapeDtypeStruct((PAIR_ROWS, 128), jnp.float32),
    )(e, w1, b1, w2, b2, w3p, b3p)


def _sage_layer(chunks, edge_index, wself, wneigh, b, relu, chunked_out):
    nc = len(chunks)
    src = edge_index[0].reshape(TILES, NSUPER, SUPER, BATCH)
    dst = edge_index[1].reshape(TILES, NSUPER, SUPER, BATCH)
    zacc = jnp.zeros((SLAB, FC), jnp.float32)
    zcnt = jnp.zeros((SLAB, 16), jnp.float32)
    ones = jnp.ones((BATCH, 16), jnp.float32)
    agg, cnt = _make_sc_aggregate(nc)(*chunks, src, dst, zacc, zcnt, ones)
    selfout = _tc_self(chunks, wself, b)
    return _tc_sage(selfout, agg, cnt, wneigh, relu, chunked_out)


def kernel(x, block0_edge_index, block1_edge_index, block2_edge_index,
           pos_edge_index, neg_edge_index,
           Wself0, Wneigh0, b0, Wself1, Wneigh1, b1, Wself2, Wneigh2, b2,
           Wd1, bd1, Wd2, bd2, Wd3, bd3):
    xc = [x[:, k * FC:(k + 1) * FC] for k in range(x.shape[1] // FC)]
    h = _sage_layer(xc, block0_edge_index, Wself0, Wneigh0, b0,
                    relu=True, chunked_out=True)
    h = _sage_layer(h, block1_edge_index, Wself1, Wneigh1, b1,
                    relu=True, chunked_out=True)
    h = _sage_layer(h, block2_edge_index, Wself2, Wneigh2, b2,
                    relu=False, chunked_out=False)[0]

    pad = jnp.zeros((480,), jnp.int32)
    gather = _make_sc_pair_gather()
    w3p = jnp.zeros((H, 128), jnp.float32).at[:, 0].set(Wd3[:, 0])
    b3p = jnp.zeros((1, 128), jnp.float32).at[0, 0].set(bd3[0])

    def half(ei):
        se = jnp.concatenate([ei[0], pad]).reshape(32, PAIR_NBATCH, PAIR_BATCH)
        de = jnp.concatenate([ei[1], pad]).reshape(32, PAIR_NBATCH, PAIR_BATCH)
        e = gather(h, se, de)
        return _tc_mlp(e, Wd1, bd1.reshape(1, H), Wd2, bd2.reshape(1, H),
                       w3p, b3p)

    h_pos = half(pos_edge_index)[:20000, 0:1]
    h_neg = half(neg_edge_index)[:20000, 0:1]
    return (h_pos, h_neg)
